# baseline (speedup 1.0000x reference)
import functools
import math

import jax
import jax.numpy as jnp
from jax import lax
from jax.experimental import pallas as pl
from jax.experimental.pallas import tpu as pltpu
from jax.experimental.pallas import tpu_sc as plsc

NUM_NODES = 100000
D_MODEL = 128
LLM_DIM = 768
B = 4096
K_NEG = 16
TOT = B + B + B * K_NEG


@functools.cache
def _make_sc_gather():
    nc, ns = 2, 16
    nw = nc * ns
    rows_w = TOT // nw
    chunk = 384
    n_chunks = rows_w // chunk

    mesh = plsc.VectorSubcoreMesh(core_axis_name="c", subcore_axis_name="s")

    @functools.partial(
        pl.kernel,
        mesh=mesh,
        out_type=(
            jax.ShapeDtypeStruct((TOT, D_MODEL), jnp.float32),
            jax.ShapeDtypeStruct((TOT, D_MODEL), jnp.float32),
        ),
        scratch_types=[
            pltpu.VMEM((rows_w,), jnp.int32),
            pltpu.VMEM((chunk, D_MODEL), jnp.float32),
            pltpu.VMEM((chunk, D_MODEL), jnp.float32),
            pltpu.SemaphoreType.DMA,
            pltpu.SemaphoreType.DMA,
        ],
    )
    def gather_k(mem_hbm, emb_hbm, idx_hbm, out_mem, out_emb,
                 idx_v, buf_a, buf_b, sem_a, sem_b):
        wid = lax.axis_index("s") * nc + lax.axis_index("c")
        base = wid * rows_w
        pltpu.sync_copy(idx_hbm.at[pl.ds(base, rows_w)], idx_v)

        def body(c, carry):
            off = c * chunk
            iv = idx_v.at[pl.ds(off, chunk)]
            ga = pltpu.make_async_copy(mem_hbm.at[iv], buf_a, sem_a)
            gb = pltpu.make_async_copy(emb_hbm.at[iv], buf_b, sem_b)
            ga.start()
            gb.start()
            ga.wait()
            gb.wait()
            wa = pltpu.make_async_copy(
                buf_a, out_mem.at[pl.ds(base + off, chunk)], sem_a)
            wb = pltpu.make_async_copy(
                buf_b, out_emb.at[pl.ds(base + off, chunk)], sem_b)
            wa.start()
            wb.start()
            wa.wait()
            wb.wait()
            return carry

        lax.fori_loop(0, n_chunks, body, 0)

    return gather_k


def _sc_gather(memory, node_emb, idx_all):
    return _make_sc_gather()(memory, node_emb, idx_all)


_SQRT_2 = math.sqrt(2.0)


def _time_enc_block(ts_col, rows):
    j = lax.broadcasted_iota(jnp.int32, (rows, D_MODEL), 1)
    half = (j // 2).astype(jnp.float32)
    div = jnp.exp(half * (-math.log(10000.0) / 64.0))
    phase = ts_col * div
    return jnp.where(j % 2 == 0, jnp.sin(phase), jnp.cos(phase))


def _ln_rows(x, g, b, eps=1e-5):
    m = jnp.mean(x, axis=-1, keepdims=True)
    v = jnp.mean((x - m) ** 2, axis=-1, keepdims=True)
    return (x - m) * lax.rsqrt(v + eps) * g + b


def _bdot(x, w):
    return jnp.dot(x.astype(jnp.bfloat16), w,
                   preferred_element_type=jnp.float32)


def _mlp_rows(x, w1, b1, w2, b2, g1, bb1, w3, b3):
    h = _bdot(x, w1) + b1
    h = 0.5 * h * (1.0 + lax.erf(h / _SQRT_2))
    h = _bdot(h, w2) + b2
    h = _ln_rows(h, g1, bb1)
    h = _bdot(h, w3) + b3
    return x + h


def _hidden_block(gm_ref, ge_ref, tenc, w1, b1, w2, b2, g1, bb1, w3, b3,
                  g2, bb2):
    x = gm_ref[...] + ge_ref[...] + tenc
    x = _ln_rows(x, g2[...], bb2[...])
    return _mlp_rows(x, w1[...], b1[...], w2[...], b2[...], g1[...], bb1[...],
                     w3[...], b3[...])


def _src_body(gm, ge, ts, w1, b1, w2, b2, g1, bb1, w3, b3, g2, bb2,
              wbil, wt, bt, u_out, heads_out, tenc_out, *, rows):
    tenc = _time_enc_block(ts[...], rows)
    tenc_out[...] = tenc
    h = _hidden_block(gm, ge, tenc, w1, b1, w2, b2, g1, bb1, w3, b3, g2, bb2)
    u_out[...] = _bdot(h, wbil[...])
    raw = _bdot(h, wt[...]) + bt[...]
    lane = lax.broadcasted_iota(jnp.int32, raw.shape, 1)
    neg_inf = jnp.float32(-jnp.inf)
    wl = jnp.where(lane < 3, raw, neg_inf)
    m = jnp.max(wl, axis=1, keepdims=True)
    e = jnp.where(lane < 3, jnp.exp(raw - m), 0.0)
    w = e / jnp.sum(e, axis=1, keepdims=True)
    sp = jnp.maximum(raw, 0.0) + jnp.log1p(jnp.exp(-jnp.abs(raw))) + 1e-6
    heads_out[...] = jnp.where(lane < 3, w, jnp.where(lane < 6, raw, sp))


def _dst_body(gm, ge, tenc_ref, u_ref, bbil, w1, b1, w2, b2, g1, bb1, w3, b3,
              g2, bb2, pos_out, *, rows):
    h = _hidden_block(gm, ge, tenc_ref[...], w1, b1, w2, b2, g1, bb1, w3, b3,
                      g2, bb2)
    pos_out[...] = jnp.sum(u_ref[...] * h, axis=1, keepdims=True) + bbil[...]


def _rep16(block, nb, rows):
    return jnp.reshape(
        jnp.broadcast_to(block[:, None, :], (nb, K_NEG, D_MODEL)),
        (rows, D_MODEL))


def _neg_body(gm, ge, tenc_ref, u_ref, bbil, w1, b1, w2, b2, g1, bb1, w3, b3,
              g2, bb2, neg_out, *, rows):
    nb = rows // K_NEG
    tenc = _rep16(tenc_ref[...], nb, rows)
    h = _hidden_block(gm, ge, tenc, w1, b1, w2, b2, g1, bb1, w3, b3, g2, bb2)
    urep = _rep16(u_ref[...], nb, rows)
    neg_out[...] = jnp.sum(urep * h, axis=1, keepdims=True) + bbil[...]


def _full(shape):
    return pl.BlockSpec(shape, lambda i: (0, 0))


def _rowblk(rows, width, off=0):
    return pl.BlockSpec((rows, width), lambda i, off=off: (i + off, 0))


def kernel(node_emb, memory, W1, b1, W2, b2, ln1_g, ln1_b, W3, b3, ln2_g,
           ln2_b, Wbil, bbil, Wt_w, bt_w, Wt_m, bt_m, Wt_s, bt_s, timestamp,
           src, dst, neg_dst):
    f32 = jnp.float32
    src = src.astype(jnp.int32)
    dst = dst.astype(jnp.int32)
    negf = neg_dst.astype(jnp.int32).reshape(-1)
    idx_all = jnp.concatenate([src, dst, negf])

    g_mem, g_emb = _sc_gather(memory, node_emb, idx_all)

    ts = timestamp.astype(f32)[:, None]

    bf16 = jnp.bfloat16
    wt = jnp.pad(jnp.concatenate([Wt_w, Wt_m, Wt_s], axis=1),
                 ((0, 0), (0, 7))).astype(bf16)
    W1, W2, W3, Wbil = (W1.astype(bf16), W2.astype(bf16), W3.astype(bf16),
                        Wbil.astype(bf16))
    bt = jnp.pad(jnp.concatenate([bt_w, bt_m, bt_s]), (0, 7))[None, :]
    b1r, b2r, b3r = b1[None, :], b2[None, :], b3[None, :]
    g1r, bb1r = ln1_g[None, :], ln1_b[None, :]
    g2r, bb2r = ln2_g[None, :], ln2_b[None, :]
    bbilr = bbil[:, None]

    R = 512
    wspecs = [
        _full((D_MODEL, LLM_DIM)),
        _full((1, LLM_DIM)),
        _full((LLM_DIM, LLM_DIM)),
        _full((1, LLM_DIM)),
        _full((1, LLM_DIM)),
        _full((1, LLM_DIM)),
        _full((LLM_DIM, D_MODEL)),
        _full((1, D_MODEL)),
        _full((1, D_MODEL)),
        _full((1, D_MODEL)),
    ]
    wargs = (W1, b1r, W2, b2r, g1r, bb1r, W3, b3r, g2r, bb2r)

    u, heads, tenc = pl.pallas_call(
        functools.partial(_src_body, rows=R),
        grid=(B // R,),
        in_specs=[
            _rowblk(R, D_MODEL),
            _rowblk(R, D_MODEL),
            _rowblk(R, 1),
            *wspecs,
            _full((D_MODEL, D_MODEL)),
            _full((D_MODEL, 16)),
            _full((1, 16)),
        ],
        out_specs=[_rowblk(R, D_MODEL), _rowblk(R, 16), _rowblk(R, D_MODEL)],
        out_shape=[
            jax.ShapeDtypeStruct((B, D_MODEL), f32),
            jax.ShapeDtypeStruct((B, 16), f32),
            jax.ShapeDtypeStruct((B, D_MODEL), f32),
        ],
    )(g_mem, g_emb, ts, *wargs, Wbil, wt, bt)

    pos = pl.pallas_call(
        functools.partial(_dst_body, rows=R),
        grid=(B // R,),
        in_specs=[
            _rowblk(R, D_MODEL, off=B // R),
            _rowblk(R, D_MODEL, off=B // R),
            _rowblk(R, D_MODEL),
            _rowblk(R, D_MODEL),
            _full((1, 1)),
            *wspecs,
        ],
        out_specs=[_rowblk(R, 1)],
        out_shape=[jax.ShapeDtypeStruct((B, 1), f32)],
    )(g_mem, g_emb, tenc, u, bbilr, *wargs)[0]

    RN = 512
    nb = RN // K_NEG
    negs = pl.pallas_call(
        functools.partial(_neg_body, rows=RN),
        grid=(B * K_NEG // RN,),
        in_specs=[
            _rowblk(RN, D_MODEL, off=2 * B // RN),
            _rowblk(RN, D_MODEL, off=2 * B // RN),
            pl.BlockSpec((nb, D_MODEL), lambda i: (i, 0)),
            pl.BlockSpec((nb, D_MODEL), lambda i: (i, 0)),
            _full((1, 1)),
            *wspecs,
        ],
        out_specs=[_rowblk(RN, 1)],
        out_shape=[jax.ShapeDtypeStruct((B * K_NEG, 1), f32)],
    )(g_mem, g_emb, tenc, u, bbilr, *wargs)[0]

    return (pos.reshape(B), negs.reshape(B, K_NEG),
            heads[:, 0:3], heads[:, 3:6], heads[:, 6:9])


# speedup vs baseline: 1.0798x; 1.0798x over previous
import functools
import math

import jax
import jax.numpy as jnp
from jax import lax
from jax.experimental import pallas as pl
from jax.experimental.pallas import tpu as pltpu
from jax.experimental.pallas import tpu_sc as plsc

NUM_NODES = 100000
D_MODEL = 128
LLM_DIM = 768
B = 4096
K_NEG = 16
TOT = B + B + B * K_NEG


@functools.cache
def _make_sc_gather():
    nc, ns = 2, 16
    nw = nc * ns
    rows_w = TOT // nw
    chunk = 384
    n_chunks = rows_w // chunk

    mesh = plsc.VectorSubcoreMesh(core_axis_name="c", subcore_axis_name="s")

    @functools.partial(
        pl.kernel,
        mesh=mesh,
        out_type=(
            jax.ShapeDtypeStruct((TOT, D_MODEL), jnp.float32),
            jax.ShapeDtypeStruct((TOT, D_MODEL), jnp.float32),
        ),
        scratch_types=[
            pltpu.VMEM((rows_w,), jnp.int32),
            pltpu.VMEM((chunk, D_MODEL), jnp.float32),
            pltpu.VMEM((chunk, D_MODEL), jnp.float32),
            pltpu.SemaphoreType.DMA,
            pltpu.SemaphoreType.DMA,
        ],
    )
    def gather_k(mem_hbm, emb_hbm, idx_hbm, out_mem, out_emb,
                 idx_v, buf_a, buf_b, sem_a, sem_b):
        wid = lax.axis_index("s") * nc + lax.axis_index("c")
        base = wid * rows_w
        pltpu.sync_copy(idx_hbm.at[pl.ds(base, rows_w)], idx_v)

        def body(c, carry):
            off = c * chunk
            iv = idx_v.at[pl.ds(off, chunk)]
            ga = pltpu.make_async_copy(mem_hbm.at[iv], buf_a, sem_a)
            gb = pltpu.make_async_copy(emb_hbm.at[iv], buf_b, sem_b)
            ga.start()
            gb.start()
            ga.wait()
            gb.wait()
            wa = pltpu.make_async_copy(
                buf_a, out_mem.at[pl.ds(base + off, chunk)], sem_a)
            wb = pltpu.make_async_copy(
                buf_b, out_emb.at[pl.ds(base + off, chunk)], sem_b)
            wa.start()
            wb.start()
            wa.wait()
            wb.wait()
            return carry

        lax.fori_loop(0, n_chunks, body, 0)

    return gather_k


def _sc_gather(memory, node_emb, idx_all):
    return _make_sc_gather()(memory, node_emb, idx_all)


_SQRT_2 = math.sqrt(2.0)


def _time_enc_block(ts_col, rows):
    j = lax.broadcasted_iota(jnp.int32, (rows, D_MODEL), 1)
    half = (j // 2).astype(jnp.float32)
    div = jnp.exp(half * (-math.log(10000.0) / 64.0))
    phase = ts_col * div
    return jnp.where(j % 2 == 0, jnp.sin(phase), jnp.cos(phase))


def _ln_rows(x, g, b, eps=1e-5):
    m = jnp.mean(x, axis=-1, keepdims=True)
    v = jnp.mean((x - m) ** 2, axis=-1, keepdims=True)
    return (x - m) * lax.rsqrt(v + eps) * g + b


def _fdot(x, w):
    return jnp.dot(x, w, preferred_element_type=jnp.float32)


def _mlp_rows(x, w1, b1, w2, b2, g1, bb1, w3, b3):
    h = _fdot(x, w1) + b1
    h = 0.5 * h * (1.0 + lax.erf(h / _SQRT_2))
    h = _fdot(h, w2) + b2
    h = _ln_rows(h, g1, bb1)
    h = _fdot(h, w3) + b3
    return x + h


def _hidden_block(gm_ref, ge_ref, tenc, w1, b1, w2, b2, g1, bb1, w3, b3,
                  g2, bb2):
    x = gm_ref[...] + ge_ref[...] + tenc
    x = _ln_rows(x, g2[...], bb2[...])
    return _mlp_rows(x, w1[...], b1[...], w2[...], b2[...], g1[...], bb1[...],
                     w3[...], b3[...])


def _src_body(gm, ge, ts, w1, b1, w2, b2, g1, bb1, w3, b3, g2, bb2,
              wbil, wt, bt, u_out, heads_out, tenc_out, *, rows):
    tenc = _time_enc_block(ts[...], rows)
    tenc_out[...] = tenc
    h = _hidden_block(gm, ge, tenc, w1, b1, w2, b2, g1, bb1, w3, b3, g2, bb2)
    u_out[...] = _fdot(h, wbil[...])
    raw = _fdot(h, wt[...]) + bt[...]
    lane = lax.broadcasted_iota(jnp.int32, raw.shape, 1)
    neg_inf = jnp.float32(-jnp.inf)
    wl = jnp.where(lane < 3, raw, neg_inf)
    m = jnp.max(wl, axis=1, keepdims=True)
    e = jnp.where(lane < 3, jnp.exp(raw - m), 0.0)
    w = e / jnp.sum(e, axis=1, keepdims=True)
    sp = jnp.maximum(raw, 0.0) + jnp.log1p(jnp.exp(-jnp.abs(raw))) + 1e-6
    heads_out[...] = jnp.where(lane < 3, w, jnp.where(lane < 6, raw, sp))


def _dst_body(gm, ge, tenc_ref, u_ref, bbil, w1, b1, w2, b2, g1, bb1, w3, b3,
              g2, bb2, pos_out, *, rows):
    h = _hidden_block(gm, ge, tenc_ref[...], w1, b1, w2, b2, g1, bb1, w3, b3,
                      g2, bb2)
    pos_out[...] = jnp.sum(u_ref[...] * h, axis=1, keepdims=True) + bbil[...]


def _rep16(block, nb, rows):
    return jnp.reshape(
        jnp.broadcast_to(block[:, None, :], (nb, K_NEG, D_MODEL)),
        (rows, D_MODEL))


def _neg_body(gm, ge, tenc_ref, u_ref, bbil, w1, b1, w2, b2, g1, bb1, w3, b3,
              g2, bb2, neg_out, *, rows):
    nb = rows // K_NEG
    tenc = _rep16(tenc_ref[...], nb, rows)
    h = _hidden_block(gm, ge, tenc, w1, b1, w2, b2, g1, bb1, w3, b3, g2, bb2)
    urep = _rep16(u_ref[...], nb, rows)
    neg_out[...] = jnp.sum(urep * h, axis=1, keepdims=True) + bbil[...]


def _full(shape):
    return pl.BlockSpec(shape, lambda i: (0, 0))


def _rowblk(rows, width, off=0):
    return pl.BlockSpec((rows, width), lambda i, off=off: (i + off, 0))


def kernel(node_emb, memory, W1, b1, W2, b2, ln1_g, ln1_b, W3, b3, ln2_g,
           ln2_b, Wbil, bbil, Wt_w, bt_w, Wt_m, bt_m, Wt_s, bt_s, timestamp,
           src, dst, neg_dst):
    f32 = jnp.float32
    src = src.astype(jnp.int32)
    dst = dst.astype(jnp.int32)
    negf = neg_dst.astype(jnp.int32).reshape(-1)
    idx_all = jnp.concatenate([src, dst, negf])

    g_mem, g_emb = _sc_gather(memory, node_emb, idx_all)

    ts = timestamp.astype(f32)[:, None]

    wt = jnp.pad(jnp.concatenate([Wt_w, Wt_m, Wt_s], axis=1),
                 ((0, 0), (0, 7)))
    bt = jnp.pad(jnp.concatenate([bt_w, bt_m, bt_s]), (0, 7))[None, :]
    b1r, b2r, b3r = b1[None, :], b2[None, :], b3[None, :]
    g1r, bb1r = ln1_g[None, :], ln1_b[None, :]
    g2r, bb2r = ln2_g[None, :], ln2_b[None, :]
    bbilr = bbil[:, None]

    R = 512
    wspecs = [
        _full((D_MODEL, LLM_DIM)),
        _full((1, LLM_DIM)),
        _full((LLM_DIM, LLM_DIM)),
        _full((1, LLM_DIM)),
        _full((1, LLM_DIM)),
        _full((1, LLM_DIM)),
        _full((LLM_DIM, D_MODEL)),
        _full((1, D_MODEL)),
        _full((1, D_MODEL)),
        _full((1, D_MODEL)),
    ]
    wargs = (W1, b1r, W2, b2r, g1r, bb1r, W3, b3r, g2r, bb2r)

    u, heads, tenc = pl.pallas_call(
        functools.partial(_src_body, rows=R),
        grid=(B // R,),
        in_specs=[
            _rowblk(R, D_MODEL),
            _rowblk(R, D_MODEL),
            _rowblk(R, 1),
            *wspecs,
            _full((D_MODEL, D_MODEL)),
            _full((D_MODEL, 16)),
            _full((1, 16)),
        ],
        out_specs=[_rowblk(R, D_MODEL), _rowblk(R, 16), _rowblk(R, D_MODEL)],
        out_shape=[
            jax.ShapeDtypeStruct((B, D_MODEL), f32),
            jax.ShapeDtypeStruct((B, 16), f32),
            jax.ShapeDtypeStruct((B, D_MODEL), f32),
        ],
    )(g_mem, g_emb, ts, *wargs, Wbil, wt, bt)

    pos = pl.pallas_call(
        functools.partial(_dst_body, rows=R),
        grid=(B // R,),
        in_specs=[
            _rowblk(R, D_MODEL, off=B // R),
            _rowblk(R, D_MODEL, off=B // R),
            _rowblk(R, D_MODEL),
            _rowblk(R, D_MODEL),
            _full((1, 1)),
            *wspecs,
        ],
        out_specs=[_rowblk(R, 1)],
        out_shape=[jax.ShapeDtypeStruct((B, 1), f32)],
    )(g_mem, g_emb, tenc, u, bbilr, *wargs)[0]

    RN = 512
    nb = RN // K_NEG
    negs = pl.pallas_call(
        functools.partial(_neg_body, rows=RN),
        grid=(B * K_NEG // RN,),
        in_specs=[
            _rowblk(RN, D_MODEL, off=2 * B // RN),
            _rowblk(RN, D_MODEL, off=2 * B // RN),
            pl.BlockSpec((nb, D_MODEL), lambda i: (i, 0)),
            pl.BlockSpec((nb, D_MODEL), lambda i: (i, 0)),
            _full((1, 1)),
            *wspecs,
        ],
        out_specs=[_rowblk(RN, 1)],
        out_shape=[jax.ShapeDtypeStruct((B * K_NEG, 1), f32)],
    )(g_mem, g_emb, tenc, u, bbilr, *wargs)[0]

    return (pos.reshape(B), negs.reshape(B, K_NEG),
            heads[:, 0:3], heads[:, 3:6], heads[:, 6:9])


# speedup vs baseline: 1.1709x; 1.0843x over previous
import functools
import math

import jax
import jax.numpy as jnp
from jax import lax
from jax.experimental import pallas as pl
from jax.experimental.pallas import tpu as pltpu
from jax.experimental.pallas import tpu_sc as plsc

NUM_NODES = 100000
D_MODEL = 128
LLM_DIM = 768
B = 4096
K_NEG = 16
TOT = B + B + B * K_NEG


@functools.cache
def _make_sc_gather(tot, chunk):
    nc, ns = 2, 16
    nw = nc * ns
    rows_w = tot // nw
    n_chunks = rows_w // chunk

    mesh = plsc.VectorSubcoreMesh(core_axis_name="c", subcore_axis_name="s")

    @functools.partial(
        pl.kernel,
        mesh=mesh,
        out_type=(
            jax.ShapeDtypeStruct((tot, D_MODEL), jnp.float32),
            jax.ShapeDtypeStruct((tot, D_MODEL), jnp.float32),
        ),
        scratch_types=[
            pltpu.VMEM((rows_w,), jnp.int32),
            pltpu.VMEM((chunk, D_MODEL), jnp.float32),
            pltpu.VMEM((chunk, D_MODEL), jnp.float32),
            pltpu.SemaphoreType.DMA,
            pltpu.SemaphoreType.DMA,
        ],
    )
    def gather_k(mem_hbm, emb_hbm, idx_hbm, out_mem, out_emb,
                 idx_v, buf_a, buf_b, sem_a, sem_b):
        wid = lax.axis_index("s") * nc + lax.axis_index("c")
        base = wid * rows_w
        pltpu.sync_copy(idx_hbm.at[pl.ds(base, rows_w)], idx_v)

        def body(c, carry):
            off = c * chunk
            iv = idx_v.at[pl.ds(off, chunk)]
            ga = pltpu.make_async_copy(mem_hbm.at[iv], buf_a, sem_a)
            gb = pltpu.make_async_copy(emb_hbm.at[iv], buf_b, sem_b)
            ga.start()
            gb.start()
            ga.wait()
            gb.wait()
            wa = pltpu.make_async_copy(
                buf_a, out_mem.at[pl.ds(base + off, chunk)], sem_a)
            wb = pltpu.make_async_copy(
                buf_b, out_emb.at[pl.ds(base + off, chunk)], sem_b)
            wa.start()
            wb.start()
            wa.wait()
            wb.wait()
            return carry

        lax.fori_loop(0, n_chunks, body, 0)

    return gather_k


def _sc_gather(memory, node_emb, idx, chunk):
    return _make_sc_gather(idx.shape[0], chunk)(memory, node_emb, idx)


_SQRT_2 = math.sqrt(2.0)


def _time_enc_block(ts_col, rows):
    j = lax.broadcasted_iota(jnp.int32, (rows, D_MODEL), 1)
    half = (j // 2).astype(jnp.float32)
    div = jnp.exp(half * (-math.log(10000.0) / 64.0))
    phase = ts_col * div
    return jnp.where(j % 2 == 0, jnp.sin(phase), jnp.cos(phase))


def _ln_rows(x, g, b, eps=1e-5):
    m = jnp.mean(x, axis=-1, keepdims=True)
    v = jnp.mean((x - m) ** 2, axis=-1, keepdims=True)
    return (x - m) * lax.rsqrt(v + eps) * g + b


def _fdot(x, w):
    return jnp.dot(x, w, preferred_element_type=jnp.float32)


def _mlp_rows(x, w1, b1, w2, b2, g1, bb1, w3, b3):
    h = _fdot(x, w1) + b1
    h = 0.5 * h * (1.0 + lax.erf(h / _SQRT_2))
    h = _fdot(h, w2) + b2
    h = _ln_rows(h, g1, bb1)
    h = _fdot(h, w3) + b3
    return x + h


def _hidden_block(gm_ref, ge_ref, tenc, w1, b1, w2, b2, g1, bb1, w3, b3,
                  g2, bb2):
    x = gm_ref[...] + ge_ref[...] + tenc
    x = _ln_rows(x, g2[...], bb2[...])
    return _mlp_rows(x, w1[...], b1[...], w2[...], b2[...], g1[...], bb1[...],
                     w3[...], b3[...])


def _src_body(gm, ge, ts, w1, b1, w2, b2, g1, bb1, w3, b3, g2, bb2,
              wbil, wt, bt, u_out, heads_out, tenc_out, *, rows):
    tenc = _time_enc_block(ts[...], rows)
    tenc_out[...] = tenc
    h = _hidden_block(gm, ge, tenc, w1, b1, w2, b2, g1, bb1, w3, b3, g2, bb2)
    u_out[...] = _fdot(h, wbil[...])
    raw = _fdot(h, wt[...]) + bt[...]
    lane = lax.broadcasted_iota(jnp.int32, raw.shape, 1)
    neg_inf = jnp.float32(-jnp.inf)
    wl = jnp.where(lane < 3, raw, neg_inf)
    m = jnp.max(wl, axis=1, keepdims=True)
    e = jnp.where(lane < 3, jnp.exp(raw - m), 0.0)
    w = e / jnp.sum(e, axis=1, keepdims=True)
    sp = jnp.maximum(raw, 0.0) + jnp.log1p(jnp.exp(-jnp.abs(raw))) + 1e-6
    heads_out[...] = jnp.where(lane < 3, w, jnp.where(lane < 6, raw, sp))


def _dst_body(gm, ge, tenc_ref, u_ref, bbil, w1, b1, w2, b2, g1, bb1, w3, b3,
              g2, bb2, pos_out, *, rows):
    h = _hidden_block(gm, ge, tenc_ref[...], w1, b1, w2, b2, g1, bb1, w3, b3,
                      g2, bb2)
    pos_out[...] = jnp.sum(u_ref[...] * h, axis=1, keepdims=True) + bbil[...]


def _rep16(block, nb, rows):
    return jnp.reshape(
        jnp.broadcast_to(block[:, None, :], (nb, K_NEG, D_MODEL)),
        (rows, D_MODEL))


def _neg_body(gm, ge, tenc_ref, u_ref, bbil, w1, b1, w2, b2, g1, bb1, w3, b3,
              g2, bb2, neg_out, *, rows):
    nb = rows // K_NEG
    tenc = _rep16(tenc_ref[...], nb, rows)
    h = _hidden_block(gm, ge, tenc, w1, b1, w2, b2, g1, bb1, w3, b3, g2, bb2)
    urep = _rep16(u_ref[...], nb, rows)
    neg_out[...] = jnp.sum(urep * h, axis=1, keepdims=True) + bbil[...]


def _full(shape):
    return pl.BlockSpec(shape, lambda i: (0, 0))


def _rowblk(rows, width, off=0):
    return pl.BlockSpec((rows, width), lambda i, off=off: (i + off, 0))


def kernel(node_emb, memory, W1, b1, W2, b2, ln1_g, ln1_b, W3, b3, ln2_g,
           ln2_b, Wbil, bbil, Wt_w, bt_w, Wt_m, bt_m, Wt_s, bt_s, timestamp,
           src, dst, neg_dst):
    f32 = jnp.float32
    src = src.astype(jnp.int32)
    dst = dst.astype(jnp.int32)
    negf = neg_dst.astype(jnp.int32).reshape(-1)

    gn_mem, gn_emb = _sc_gather(memory, node_emb, negf, 256)
    idx_sd = jnp.concatenate([src, dst])
    g_mem, g_emb = _sc_gather(memory, node_emb, idx_sd, 256)

    ts = timestamp.astype(f32)[:, None]

    wt = jnp.pad(jnp.concatenate([Wt_w, Wt_m, Wt_s], axis=1),
                 ((0, 0), (0, 7)))
    bt = jnp.pad(jnp.concatenate([bt_w, bt_m, bt_s]), (0, 7))[None, :]
    b1r, b2r, b3r = b1[None, :], b2[None, :], b3[None, :]
    g1r, bb1r = ln1_g[None, :], ln1_b[None, :]
    g2r, bb2r = ln2_g[None, :], ln2_b[None, :]
    bbilr = bbil[:, None]

    R = 512
    wspecs = [
        _full((D_MODEL, LLM_DIM)),
        _full((1, LLM_DIM)),
        _full((LLM_DIM, LLM_DIM)),
        _full((1, LLM_DIM)),
        _full((1, LLM_DIM)),
        _full((1, LLM_DIM)),
        _full((LLM_DIM, D_MODEL)),
        _full((1, D_MODEL)),
        _full((1, D_MODEL)),
        _full((1, D_MODEL)),
    ]
    wargs = (W1, b1r, W2, b2r, g1r, bb1r, W3, b3r, g2r, bb2r)

    u, heads, tenc = pl.pallas_call(
        functools.partial(_src_body, rows=R),
        grid=(B // R,),
        in_specs=[
            _rowblk(R, D_MODEL),
            _rowblk(R, D_MODEL),
            _rowblk(R, 1),
            *wspecs,
            _full((D_MODEL, D_MODEL)),
            _full((D_MODEL, 16)),
            _full((1, 16)),
        ],
        out_specs=[_rowblk(R, D_MODEL), _rowblk(R, 16), _rowblk(R, D_MODEL)],
        out_shape=[
            jax.ShapeDtypeStruct((B, D_MODEL), f32),
            jax.ShapeDtypeStruct((B, 16), f32),
            jax.ShapeDtypeStruct((B, D_MODEL), f32),
        ],
    )(g_mem, g_emb, ts, *wargs, Wbil, wt, bt)

    pos = pl.pallas_call(
        functools.partial(_dst_body, rows=R),
        grid=(B // R,),
        in_specs=[
            _rowblk(R, D_MODEL, off=B // R),
            _rowblk(R, D_MODEL, off=B // R),
            _rowblk(R, D_MODEL),
            _rowblk(R, D_MODEL),
            _full((1, 1)),
            *wspecs,
        ],
        out_specs=[_rowblk(R, 1)],
        out_shape=[jax.ShapeDtypeStruct((B, 1), f32)],
    )(g_mem, g_emb, tenc, u, bbilr, *wargs)[0]

    RN = 512
    nb = RN // K_NEG
    negs = pl.pallas_call(
        functools.partial(_neg_body, rows=RN),
        grid=(B * K_NEG // RN,),
        in_specs=[
            _rowblk(RN, D_MODEL),
            _rowblk(RN, D_MODEL),
            pl.BlockSpec((nb, D_MODEL), lambda i: (i, 0)),
            pl.BlockSpec((nb, D_MODEL), lambda i: (i, 0)),
            _full((1, 1)),
            *wspecs,
        ],
        out_specs=[_rowblk(RN, 1)],
        out_shape=[jax.ShapeDtypeStruct((B * K_NEG, 1), f32)],
    )(gn_mem, gn_emb, tenc, u, bbilr, *wargs)[0]

    return (pos.reshape(B), negs.reshape(B, K_NEG),
            heads[:, 0:3], heads[:, 3:6], heads[:, 6:9])


# speedup vs baseline: 1.2868x; 1.0990x over previous
import functools
import math

import jax
import jax.numpy as jnp
from jax import lax
from jax.experimental import pallas as pl
from jax.experimental.pallas import tpu as pltpu
from jax.experimental.pallas import tpu_sc as plsc

NUM_NODES = 100000
D_MODEL = 128
LLM_DIM = 768
B = 4096
K_NEG = 16
TOT = B + B + B * K_NEG


@functools.cache
def _make_sc_gather(tot, chunk):
    nc, ns = 2, 16
    nw = nc * ns
    rows_w = tot // nw
    n_chunks = rows_w // chunk

    mesh = plsc.VectorSubcoreMesh(core_axis_name="c", subcore_axis_name="s")

    @functools.partial(
        pl.kernel,
        mesh=mesh,
        out_type=(
            jax.ShapeDtypeStruct((tot, D_MODEL), jnp.float32),
            jax.ShapeDtypeStruct((tot, D_MODEL), jnp.float32),
        ),
        scratch_types=[
            pltpu.VMEM((rows_w,), jnp.int32),
            pltpu.VMEM((chunk, D_MODEL), jnp.float32),
            pltpu.VMEM((chunk, D_MODEL), jnp.float32),
            pltpu.SemaphoreType.DMA,
            pltpu.SemaphoreType.DMA,
        ],
    )
    def gather_k(mem_hbm, emb_hbm, idx_hbm, out_mem, out_emb,
                 idx_v, buf_a, buf_b, sem_a, sem_b):
        wid = lax.axis_index("s") * nc + lax.axis_index("c")
        base = wid * rows_w
        pltpu.sync_copy(idx_hbm.at[pl.ds(base, rows_w)], idx_v)

        def body(c, carry):
            off = c * chunk
            iv = idx_v.at[pl.ds(off, chunk)]
            ga = pltpu.make_async_copy(mem_hbm.at[iv], buf_a, sem_a)
            gb = pltpu.make_async_copy(emb_hbm.at[iv], buf_b, sem_b)
            ga.start()
            gb.start()
            ga.wait()
            gb.wait()
            wa = pltpu.make_async_copy(
                buf_a, out_mem.at[pl.ds(base + off, chunk)], sem_a)
            wb = pltpu.make_async_copy(
                buf_b, out_emb.at[pl.ds(base + off, chunk)], sem_b)
            wa.start()
            wb.start()
            wa.wait()
            wb.wait()
            return carry

        lax.fori_loop(0, n_chunks, body, 0)

    return gather_k


def _sc_gather(memory, node_emb, idx, chunk):
    return _make_sc_gather(idx.shape[0], chunk)(memory, node_emb, idx)


_SQRT_2 = math.sqrt(2.0)


def _time_enc_block(ts_col, rows):
    j = lax.broadcasted_iota(jnp.int32, (rows, D_MODEL), 1)
    half = (j // 2).astype(jnp.float32)
    div = jnp.exp(half * (-math.log(10000.0) / 64.0))
    phase = ts_col * div
    return jnp.where(j % 2 == 0, jnp.sin(phase), jnp.cos(phase))


def _ln_rows(x, g, b, eps=1e-5):
    m = jnp.mean(x, axis=-1, keepdims=True)
    v = jnp.mean((x - m) ** 2, axis=-1, keepdims=True)
    return (x - m) * lax.rsqrt(v + eps) * g + b


def _fdot(x, w):
    return jnp.dot(x, w, preferred_element_type=jnp.float32)


def _mlp_rows(x, w1, b1, w2, b2, g1, bb1, w3, b3):
    h = _fdot(x, w1) + b1
    h = 0.5 * h * (1.0 + lax.erf(h / _SQRT_2))
    h = _fdot(h, w2) + b2
    h = _ln_rows(h, g1, bb1)
    h = _fdot(h, w3) + b3
    return x + h


def _hidden_block(gm_ref, ge_ref, tenc, w1, b1, w2, b2, g1, bb1, w3, b3,
                  g2, bb2):
    x = gm_ref[...] + ge_ref[...] + tenc
    x = _ln_rows(x, g2[...], bb2[...])
    return _mlp_rows(x, w1[...], b1[...], w2[...], b2[...], g1[...], bb1[...],
                     w3[...], b3[...])


def _src_body(gm, ge, ts, w1, b1, w2, b2, g1, bb1, w3, b3, g2, bb2,
              wbil, wt, bt, u_out, heads_out, tenc_out, *, rows):
    tenc = _time_enc_block(ts[...], rows)
    tenc_out[...] = tenc
    h = _hidden_block(gm, ge, tenc, w1, b1, w2, b2, g1, bb1, w3, b3, g2, bb2)
    u_out[...] = _fdot(h, wbil[...])
    raw = _fdot(h, wt[...]) + bt[...]
    lane = lax.broadcasted_iota(jnp.int32, raw.shape, 1)
    neg_inf = jnp.float32(-jnp.inf)
    wl = jnp.where(lane < 3, raw, neg_inf)
    m = jnp.max(wl, axis=1, keepdims=True)
    e = jnp.where(lane < 3, jnp.exp(raw - m), 0.0)
    w = e / jnp.sum(e, axis=1, keepdims=True)
    sp = jnp.maximum(raw, 0.0) + jnp.log1p(jnp.exp(-jnp.abs(raw))) + 1e-6
    heads_out[...] = jnp.where(lane < 3, w, jnp.where(lane < 6, raw, sp))


def _dst_body(gm, ge, tenc_ref, u_ref, bbil, w1, b1, w2, b2, g1, bb1, w3, b3,
              g2, bb2, pos_out, *, rows):
    h = _hidden_block(gm, ge, tenc_ref[...], w1, b1, w2, b2, g1, bb1, w3, b3,
                      g2, bb2)
    pos_out[...] = jnp.sum(u_ref[...] * h, axis=1, keepdims=True) + bbil[...]


def _rep16(block, nb, rows):
    return jnp.reshape(
        jnp.broadcast_to(block[:, None, :], (nb, K_NEG, D_MODEL)),
        (rows, D_MODEL))


def _neg_body(gm, ge, tenc_ref, u_ref, bbil, w1, b1, w2, b2, g1, bb1, w3, b3,
              g2, bb2, neg_out, *, rows):
    nb = rows // K_NEG
    tenc = _rep16(tenc_ref[...], nb, rows)
    h = _hidden_block(gm, ge, tenc, w1, b1, w2, b2, g1, bb1, w3, b3, g2, bb2)
    urep = _rep16(u_ref[...], nb, rows)
    neg_out[...] = jnp.sum(urep * h, axis=1, keepdims=True) + bbil[...]


def _full(shape):
    return pl.BlockSpec(shape, lambda i: (0, 0))


def _rowblk(rows, width, off=0):
    return pl.BlockSpec((rows, width), lambda i, off=off: (i + off, 0))


def kernel(node_emb, memory, W1, b1, W2, b2, ln1_g, ln1_b, W3, b3, ln2_g,
           ln2_b, Wbil, bbil, Wt_w, bt_w, Wt_m, bt_m, Wt_s, bt_s, timestamp,
           src, dst, neg_dst):
    f32 = jnp.float32
    src = src.astype(jnp.int32)
    dst = dst.astype(jnp.int32)
    negf = neg_dst.astype(jnp.int32).reshape(-1)

    gn_mem, gn_emb = _sc_gather(memory, node_emb, negf, 256)
    idx_sd = jnp.concatenate([src, dst])
    g_mem, g_emb = _sc_gather(memory, node_emb, idx_sd, 256)

    ts = timestamp.astype(f32)[:, None]

    wt = jnp.pad(jnp.concatenate([Wt_w, Wt_m, Wt_s], axis=1),
                 ((0, 0), (0, 7)))
    bt = jnp.pad(jnp.concatenate([bt_w, bt_m, bt_s]), (0, 7))[None, :]
    b1r, b2r, b3r = b1[None, :], b2[None, :], b3[None, :]
    g1r, bb1r = ln1_g[None, :], ln1_b[None, :]
    g2r, bb2r = ln2_g[None, :], ln2_b[None, :]
    bbilr = bbil[:, None]

    R = 512
    wspecs = [
        _full((D_MODEL, LLM_DIM)),
        _full((1, LLM_DIM)),
        _full((LLM_DIM, LLM_DIM)),
        _full((1, LLM_DIM)),
        _full((1, LLM_DIM)),
        _full((1, LLM_DIM)),
        _full((LLM_DIM, D_MODEL)),
        _full((1, D_MODEL)),
        _full((1, D_MODEL)),
        _full((1, D_MODEL)),
    ]
    wargs = (W1, b1r, W2, b2r, g1r, bb1r, W3, b3r, g2r, bb2r)

    u, heads, tenc = pl.pallas_call(
        functools.partial(_src_body, rows=R),
        grid=(B // R,),
        in_specs=[
            _rowblk(R, D_MODEL),
            _rowblk(R, D_MODEL),
            _rowblk(R, 1),
            *wspecs,
            _full((D_MODEL, D_MODEL)),
            _full((D_MODEL, 16)),
            _full((1, 16)),
        ],
        out_specs=[_rowblk(R, D_MODEL), _rowblk(R, 16), _rowblk(R, D_MODEL)],
        out_shape=[
            jax.ShapeDtypeStruct((B, D_MODEL), f32),
            jax.ShapeDtypeStruct((B, 16), f32),
            jax.ShapeDtypeStruct((B, D_MODEL), f32),
        ],
    )(g_mem, g_emb, ts, *wargs, Wbil, wt, bt)

    pos = pl.pallas_call(
        functools.partial(_dst_body, rows=R),
        grid=(B // R,),
        in_specs=[
            _rowblk(R, D_MODEL, off=B // R),
            _rowblk(R, D_MODEL, off=B // R),
            _rowblk(R, D_MODEL),
            _rowblk(R, D_MODEL),
            _full((1, 1)),
            *wspecs,
        ],
        out_specs=[_rowblk(R, 1)],
        out_shape=[jax.ShapeDtypeStruct((B, 1), f32)],
    )(g_mem, g_emb, tenc, u, bbilr, *wargs)[0]

    RN = 1024
    nb = RN // K_NEG
    negs = pl.pallas_call(
        functools.partial(_neg_body, rows=RN),
        grid=(B * K_NEG // RN,),
        in_specs=[
            _rowblk(RN, D_MODEL),
            _rowblk(RN, D_MODEL),
            pl.BlockSpec((nb, D_MODEL), lambda i: (i, 0)),
            pl.BlockSpec((nb, D_MODEL), lambda i: (i, 0)),
            _full((1, 1)),
            *wspecs,
        ],
        out_specs=[_rowblk(RN, 1)],
        out_shape=[jax.ShapeDtypeStruct((B * K_NEG, 1), f32)],
    )(gn_mem, gn_emb, tenc, u, bbilr, *wargs)[0]

    return (pos.reshape(B), negs.reshape(B, K_NEG),
            heads[:, 0:3], heads[:, 3:6], heads[:, 6:9])


# speedup vs baseline: 1.3428x; 1.0436x over previous
import functools
import math

import jax
import jax.numpy as jnp
from jax import lax
from jax.experimental import pallas as pl
from jax.experimental.pallas import tpu as pltpu
from jax.experimental.pallas import tpu_sc as plsc

NUM_NODES = 100000
D_MODEL = 128
LLM_DIM = 768
B = 4096
K_NEG = 16
TOT = B + B + B * K_NEG


@functools.cache
def _make_sc_gather(tot, chunk):
    nc, ns = 2, 16
    nw = nc * ns
    rows_w = tot // nw
    n_chunks = rows_w // chunk

    mesh = plsc.VectorSubcoreMesh(core_axis_name="c", subcore_axis_name="s")

    @functools.partial(
        pl.kernel,
        mesh=mesh,
        out_type=(
            jax.ShapeDtypeStruct((tot, D_MODEL), jnp.float32),
            jax.ShapeDtypeStruct((tot, D_MODEL), jnp.float32),
        ),
        scratch_types=[
            pltpu.VMEM((rows_w,), jnp.int32),
            pltpu.VMEM((chunk, D_MODEL), jnp.float32),
            pltpu.VMEM((chunk, D_MODEL), jnp.float32),
            pltpu.SemaphoreType.DMA,
            pltpu.SemaphoreType.DMA,
        ],
    )
    def gather_k(mem_hbm, emb_hbm, idx_hbm, out_mem, out_emb,
                 idx_v, buf_a, buf_b, sem_a, sem_b):
        wid = lax.axis_index("s") * nc + lax.axis_index("c")
        base = wid * rows_w
        pltpu.sync_copy(idx_hbm.at[pl.ds(base, rows_w)], idx_v)

        def body(c, carry):
            off = c * chunk
            iv = idx_v.at[pl.ds(off, chunk)]
            ga = pltpu.make_async_copy(mem_hbm.at[iv], buf_a, sem_a)
            gb = pltpu.make_async_copy(emb_hbm.at[iv], buf_b, sem_b)
            ga.start()
            gb.start()
            ga.wait()
            gb.wait()
            wa = pltpu.make_async_copy(
                buf_a, out_mem.at[pl.ds(base + off, chunk)], sem_a)
            wb = pltpu.make_async_copy(
                buf_b, out_emb.at[pl.ds(base + off, chunk)], sem_b)
            wa.start()
            wb.start()
            wa.wait()
            wb.wait()
            return carry

        lax.fori_loop(0, n_chunks, body, 0)

    return gather_k


def _sc_gather(memory, node_emb, idx, chunk):
    return _make_sc_gather(idx.shape[0], chunk)(memory, node_emb, idx)


_SQRT_2 = math.sqrt(2.0)


def _time_enc_block(ts_col, rows):
    j = lax.broadcasted_iota(jnp.int32, (rows, D_MODEL), 1)
    half = (j // 2).astype(jnp.float32)
    div = jnp.exp(half * (-math.log(10000.0) / 64.0))
    phase = ts_col * div
    return jnp.where(j % 2 == 0, jnp.sin(phase), jnp.cos(phase))


def _ln_rows(x, g, b, eps=1e-5):
    m = jnp.mean(x, axis=-1, keepdims=True)
    v = jnp.mean((x - m) ** 2, axis=-1, keepdims=True)
    return (x - m) * lax.rsqrt(v + eps) * g + b


def _fdot(x, w):
    return jnp.dot(x, w, preferred_element_type=jnp.float32)


def _mlp_rows(x, w1, b1, w2, b2, g1, bb1, w3, b3):
    h = _fdot(x, w1) + b1
    h = 0.5 * h * (1.0 + lax.erf(h / _SQRT_2))
    h = _fdot(h, w2) + b2
    h = _ln_rows(h, g1, bb1)
    h = _fdot(h, w3) + b3
    return x + h


def _hidden_block(gm_ref, ge_ref, tenc, w1, b1, w2, b2, g1, bb1, w3, b3,
                  g2, bb2):
    x = gm_ref[...] + ge_ref[...] + tenc
    x = _ln_rows(x, g2[...], bb2[...])
    return _mlp_rows(x, w1[...], b1[...], w2[...], b2[...], g1[...], bb1[...],
                     w3[...], b3[...])


def _src_body(gm, ge, ts, w1, b1, w2, b2, g1, bb1, w3, b3, g2, bb2,
              wbil, wt, bt, u_out, heads_out, tenc_out, *, rows):
    tenc = _time_enc_block(ts[...], rows)
    tenc_out[...] = tenc
    h = _hidden_block(gm, ge, tenc, w1, b1, w2, b2, g1, bb1, w3, b3, g2, bb2)
    u_out[...] = _fdot(h, wbil[...])
    raw = _fdot(h, wt[...]) + bt[...]
    lane = lax.broadcasted_iota(jnp.int32, raw.shape, 1)
    neg_inf = jnp.float32(-jnp.inf)
    wl = jnp.where(lane < 3, raw, neg_inf)
    m = jnp.max(wl, axis=1, keepdims=True)
    e = jnp.where(lane < 3, jnp.exp(raw - m), 0.0)
    w = e / jnp.sum(e, axis=1, keepdims=True)
    sp = jnp.maximum(raw, 0.0) + jnp.log1p(jnp.exp(-jnp.abs(raw))) + 1e-6
    heads_out[...] = jnp.where(lane < 3, w, jnp.where(lane < 6, raw, sp))


def _dst_body(gm, ge, tenc_ref, u_ref, bbil, w1, b1, w2, b2, g1, bb1, w3, b3,
              g2, bb2, pos_out, *, rows):
    h = _hidden_block(gm, ge, tenc_ref[...], w1, b1, w2, b2, g1, bb1, w3, b3,
                      g2, bb2)
    pos_out[...] = jnp.sum(u_ref[...] * h, axis=1, keepdims=True) + bbil[...]


def _rep16(block, nb, rows):
    return jnp.reshape(
        jnp.broadcast_to(block[:, None, :], (nb, K_NEG, D_MODEL)),
        (rows, D_MODEL))


def _neg_body(gm, ge, tenc_ref, u_ref, bbil, w1, b1, w2, b2, g1, bb1, w3, b3,
              g2, bb2, neg_out, *, rows):
    nb = rows // K_NEG
    tenc = _rep16(tenc_ref[...], nb, rows)
    h = _hidden_block(gm, ge, tenc, w1, b1, w2, b2, g1, bb1, w3, b3, g2, bb2)
    urep = _rep16(u_ref[...], nb, rows)
    neg_out[...] = jnp.sum(urep * h, axis=1, keepdims=True) + bbil[...]


def _full(shape):
    return pl.BlockSpec(shape, lambda i: (0, 0))


def _rowblk(rows, width, off=0):
    return pl.BlockSpec((rows, width), lambda i, off=off: (i + off, 0))


def kernel(node_emb, memory, W1, b1, W2, b2, ln1_g, ln1_b, W3, b3, ln2_g,
           ln2_b, Wbil, bbil, Wt_w, bt_w, Wt_m, bt_m, Wt_s, bt_s, timestamp,
           src, dst, neg_dst):
    f32 = jnp.float32
    src = src.astype(jnp.int32)
    dst = dst.astype(jnp.int32)
    negf = neg_dst.astype(jnp.int32).reshape(-1)

    gn_mem, gn_emb = _sc_gather(memory, node_emb, negf, 256)
    idx_sd = jnp.concatenate([src, dst])
    g_mem, g_emb = _sc_gather(memory, node_emb, idx_sd, 256)

    ts = timestamp.astype(f32)[:, None]

    wt = jnp.pad(jnp.concatenate([Wt_w, Wt_m, Wt_s], axis=1),
                 ((0, 0), (0, 7)))
    bt = jnp.pad(jnp.concatenate([bt_w, bt_m, bt_s]), (0, 7))[None, :]
    b1r, b2r, b3r = b1[None, :], b2[None, :], b3[None, :]
    g1r, bb1r = ln1_g[None, :], ln1_b[None, :]
    g2r, bb2r = ln2_g[None, :], ln2_b[None, :]
    bbilr = bbil[:, None]

    R = 1024
    wspecs = [
        _full((D_MODEL, LLM_DIM)),
        _full((1, LLM_DIM)),
        _full((LLM_DIM, LLM_DIM)),
        _full((1, LLM_DIM)),
        _full((1, LLM_DIM)),
        _full((1, LLM_DIM)),
        _full((LLM_DIM, D_MODEL)),
        _full((1, D_MODEL)),
        _full((1, D_MODEL)),
        _full((1, D_MODEL)),
    ]
    wargs = (W1, b1r, W2, b2r, g1r, bb1r, W3, b3r, g2r, bb2r)

    u, heads, tenc = pl.pallas_call(
        functools.partial(_src_body, rows=R),
        grid=(B // R,),
        in_specs=[
            _rowblk(R, D_MODEL),
            _rowblk(R, D_MODEL),
            _rowblk(R, 1),
            *wspecs,
            _full((D_MODEL, D_MODEL)),
            _full((D_MODEL, 16)),
            _full((1, 16)),
        ],
        out_specs=[_rowblk(R, D_MODEL), _rowblk(R, 16), _rowblk(R, D_MODEL)],
        out_shape=[
            jax.ShapeDtypeStruct((B, D_MODEL), f32),
            jax.ShapeDtypeStruct((B, 16), f32),
            jax.ShapeDtypeStruct((B, D_MODEL), f32),
        ],
    )(g_mem, g_emb, ts, *wargs, Wbil, wt, bt)

    pos = pl.pallas_call(
        functools.partial(_dst_body, rows=R),
        grid=(B // R,),
        in_specs=[
            _rowblk(R, D_MODEL, off=B // R),
            _rowblk(R, D_MODEL, off=B // R),
            _rowblk(R, D_MODEL),
            _rowblk(R, D_MODEL),
            _full((1, 1)),
            *wspecs,
        ],
        out_specs=[_rowblk(R, 1)],
        out_shape=[jax.ShapeDtypeStruct((B, 1), f32)],
    )(g_mem, g_emb, tenc, u, bbilr, *wargs)[0]

    RN = 2048
    nb = RN // K_NEG
    negs = pl.pallas_call(
        functools.partial(_neg_body, rows=RN),
        grid=(B * K_NEG // RN,),
        in_specs=[
            _rowblk(RN, D_MODEL),
            _rowblk(RN, D_MODEL),
            pl.BlockSpec((nb, D_MODEL), lambda i: (i, 0)),
            pl.BlockSpec((nb, D_MODEL), lambda i: (i, 0)),
            _full((1, 1)),
            *wspecs,
        ],
        out_specs=[_rowblk(RN, 1)],
        out_shape=[jax.ShapeDtypeStruct((B * K_NEG, 1), f32)],
    )(gn_mem, gn_emb, tenc, u, bbilr, *wargs)[0]

    return (pos.reshape(B), negs.reshape(B, K_NEG),
            heads[:, 0:3], heads[:, 3:6], heads[:, 6:9])


# speedup vs baseline: 1.3750x; 1.0239x over previous
import functools
import math

import jax
import jax.numpy as jnp
from jax import lax
from jax.experimental import pallas as pl
from jax.experimental.pallas import tpu as pltpu
from jax.experimental.pallas import tpu_sc as plsc

NUM_NODES = 100000
D_MODEL = 128
LLM_DIM = 768
B = 4096
K_NEG = 16
TOT = B + B + B * K_NEG


@functools.cache
def _make_sc_gather(tot, chunk):
    nc, ns = 2, 16
    nw = nc * ns
    rows_w = tot // nw
    n_chunks = rows_w // chunk

    mesh = plsc.VectorSubcoreMesh(core_axis_name="c", subcore_axis_name="s")

    @functools.partial(
        pl.kernel,
        mesh=mesh,
        out_type=(
            jax.ShapeDtypeStruct((tot, D_MODEL), jnp.float32),
            jax.ShapeDtypeStruct((tot, D_MODEL), jnp.float32),
        ),
        scratch_types=[
            pltpu.VMEM((rows_w,), jnp.int32),
            pltpu.VMEM((chunk, D_MODEL), jnp.float32),
            pltpu.VMEM((chunk, D_MODEL), jnp.float32),
            pltpu.SemaphoreType.DMA,
            pltpu.SemaphoreType.DMA,
        ],
    )
    def gather_k(mem_hbm, emb_hbm, idx_hbm, out_mem, out_emb,
                 idx_v, buf_a, buf_b, sem_a, sem_b):
        wid = lax.axis_index("s") * nc + lax.axis_index("c")
        base = wid * rows_w
        pltpu.sync_copy(idx_hbm.at[pl.ds(base, rows_w)], idx_v)

        def body(c, carry):
            off = c * chunk
            iv = idx_v.at[pl.ds(off, chunk)]
            ga = pltpu.make_async_copy(mem_hbm.at[iv], buf_a, sem_a)
            gb = pltpu.make_async_copy(emb_hbm.at[iv], buf_b, sem_b)
            ga.start()
            gb.start()
            ga.wait()
            gb.wait()
            wa = pltpu.make_async_copy(
                buf_a, out_mem.at[pl.ds(base + off, chunk)], sem_a)
            wb = pltpu.make_async_copy(
                buf_b, out_emb.at[pl.ds(base + off, chunk)], sem_b)
            wa.start()
            wb.start()
            wa.wait()
            wb.wait()
            return carry

        lax.fori_loop(0, n_chunks, body, 0)

    return gather_k


def _sc_gather(memory, node_emb, idx, chunk):
    return _make_sc_gather(idx.shape[0], chunk)(memory, node_emb, idx)


_SQRT_2 = math.sqrt(2.0)


def _time_enc_block(ts_col, rows):
    j = lax.broadcasted_iota(jnp.int32, (rows, D_MODEL), 1)
    half = (j // 2).astype(jnp.float32)
    div = jnp.exp(half * (-math.log(10000.0) / 64.0))
    phase = ts_col * div
    return jnp.where(j % 2 == 0, jnp.sin(phase), jnp.cos(phase))


def _ln_rows(x, g, b, eps=1e-5):
    m = jnp.mean(x, axis=-1, keepdims=True)
    v = jnp.mean((x - m) ** 2, axis=-1, keepdims=True)
    return (x - m) * lax.rsqrt(v + eps) * g + b


def _fdot(x, w):
    return jnp.dot(x, w, preferred_element_type=jnp.float32)


def _mlp_rows(x, w1, b1, w2, b2, g1, bb1, w3, b3):
    h = _fdot(x, w1) + b1
    h = 0.5 * h * (1.0 + lax.erf(h / _SQRT_2))
    h = _fdot(h, w2) + b2
    h = _ln_rows(h, g1, bb1)
    h = _fdot(h, w3) + b3
    return x + h


def _hidden_block(gm_ref, ge_ref, tenc, w1, b1, w2, b2, g1, bb1, w3, b3,
                  g2, bb2):
    x = gm_ref[...] + ge_ref[...] + tenc
    x = _ln_rows(x, g2[...], bb2[...])
    return _mlp_rows(x, w1[...], b1[...], w2[...], b2[...], g1[...], bb1[...],
                     w3[...], b3[...])


def _src_body(gm, ge, ts, w1, b1, w2, b2, g1, bb1, w3, b3, g2, bb2,
              wbil, wt, bt, u_out, heads_out, tenc_out, *, rows):
    tenc = _time_enc_block(ts[...], rows)
    tenc_out[...] = tenc
    h = _hidden_block(gm, ge, tenc, w1, b1, w2, b2, g1, bb1, w3, b3, g2, bb2)
    u_out[...] = _fdot(h, wbil[...])
    raw = _fdot(h, wt[...]) + bt[...]
    lane = lax.broadcasted_iota(jnp.int32, raw.shape, 1)
    neg_inf = jnp.float32(-jnp.inf)
    wl = jnp.where(lane < 3, raw, neg_inf)
    m = jnp.max(wl, axis=1, keepdims=True)
    e = jnp.where(lane < 3, jnp.exp(raw - m), 0.0)
    w = e / jnp.sum(e, axis=1, keepdims=True)
    sp = jnp.maximum(raw, 0.0) + jnp.log1p(jnp.exp(-jnp.abs(raw))) + 1e-6
    heads_out[...] = jnp.where(lane < 3, w, jnp.where(lane < 6, raw, sp))


def _dst_body(gm, ge, tenc_ref, u_ref, bbil, w1, b1, w2, b2, g1, bb1, w3, b3,
              g2, bb2, pos_out, *, rows):
    h = _hidden_block(gm, ge, tenc_ref[...], w1, b1, w2, b2, g1, bb1, w3, b3,
                      g2, bb2)
    pos_out[...] = jnp.sum(u_ref[...] * h, axis=1, keepdims=True) + bbil[...]


def _rep16(block, nb, rows):
    return jnp.reshape(
        jnp.broadcast_to(block[:, None, :], (nb, K_NEG, D_MODEL)),
        (rows, D_MODEL))


def _neg_body(gm, ge, tenc_ref, u_ref, bbil, w1, b1, w2, b2, g1, bb1, w3, b3,
              g2, bb2, neg_out, *, rows):
    nb = rows // K_NEG
    tenc = _rep16(tenc_ref[...], nb, rows)
    h = _hidden_block(gm, ge, tenc, w1, b1, w2, b2, g1, bb1, w3, b3, g2, bb2)
    urep = _rep16(u_ref[...], nb, rows)
    neg_out[...] = jnp.sum(urep * h, axis=1, keepdims=True) + bbil[...]


def _full(shape):
    return pl.BlockSpec(shape, lambda i: (0, 0))


def _rowblk(rows, width, off=0):
    return pl.BlockSpec((rows, width), lambda i, off=off: (i + off, 0))


def kernel(node_emb, memory, W1, b1, W2, b2, ln1_g, ln1_b, W3, b3, ln2_g,
           ln2_b, Wbil, bbil, Wt_w, bt_w, Wt_m, bt_m, Wt_s, bt_s, timestamp,
           src, dst, neg_dst):
    f32 = jnp.float32
    src = src.astype(jnp.int32)
    dst = dst.astype(jnp.int32)
    negf = neg_dst.astype(jnp.int32).reshape(-1)

    gn_mem, gn_emb = _sc_gather(memory, node_emb, negf, 256)
    idx_sd = jnp.concatenate([src, dst])
    g_mem, g_emb = _sc_gather(memory, node_emb, idx_sd, 256)

    ts = timestamp.astype(f32)[:, None]

    wt = jnp.pad(jnp.concatenate([Wt_w, Wt_m, Wt_s], axis=1),
                 ((0, 0), (0, 7)))
    bt = jnp.pad(jnp.concatenate([bt_w, bt_m, bt_s]), (0, 7))[None, :]
    b1r, b2r, b3r = b1[None, :], b2[None, :], b3[None, :]
    g1r, bb1r = ln1_g[None, :], ln1_b[None, :]
    g2r, bb2r = ln2_g[None, :], ln2_b[None, :]
    bbilr = bbil[:, None]

    R = 1024
    wspecs = [
        _full((D_MODEL, LLM_DIM)),
        _full((1, LLM_DIM)),
        _full((LLM_DIM, LLM_DIM)),
        _full((1, LLM_DIM)),
        _full((1, LLM_DIM)),
        _full((1, LLM_DIM)),
        _full((LLM_DIM, D_MODEL)),
        _full((1, D_MODEL)),
        _full((1, D_MODEL)),
        _full((1, D_MODEL)),
    ]
    wargs = (W1, b1r, W2, b2r, g1r, bb1r, W3, b3r, g2r, bb2r)

    u, heads, tenc = pl.pallas_call(
        functools.partial(_src_body, rows=R),
        grid=(B // R,),
        in_specs=[
            _rowblk(R, D_MODEL),
            _rowblk(R, D_MODEL),
            _rowblk(R, 1),
            *wspecs,
            _full((D_MODEL, D_MODEL)),
            _full((D_MODEL, 16)),
            _full((1, 16)),
        ],
        out_specs=[_rowblk(R, D_MODEL), _rowblk(R, 16), _rowblk(R, D_MODEL)],
        out_shape=[
            jax.ShapeDtypeStruct((B, D_MODEL), f32),
            jax.ShapeDtypeStruct((B, 16), f32),
            jax.ShapeDtypeStruct((B, D_MODEL), f32),
        ],
    )(g_mem, g_emb, ts, *wargs, Wbil, wt, bt)

    pos = pl.pallas_call(
        functools.partial(_dst_body, rows=R),
        grid=(B // R,),
        in_specs=[
            _rowblk(R, D_MODEL, off=B // R),
            _rowblk(R, D_MODEL, off=B // R),
            _rowblk(R, D_MODEL),
            _rowblk(R, D_MODEL),
            _full((1, 1)),
            *wspecs,
        ],
        out_specs=[_rowblk(R, 1)],
        out_shape=[jax.ShapeDtypeStruct((B, 1), f32)],
    )(g_mem, g_emb, tenc, u, bbilr, *wargs)[0]

    RN = 4096
    nb = RN // K_NEG
    negs = pl.pallas_call(
        functools.partial(_neg_body, rows=RN),
        grid=(B * K_NEG // RN,),
        in_specs=[
            _rowblk(RN, D_MODEL),
            _rowblk(RN, D_MODEL),
            pl.BlockSpec((nb, D_MODEL), lambda i: (i, 0)),
            pl.BlockSpec((nb, D_MODEL), lambda i: (i, 0)),
            _full((1, 1)),
            *wspecs,
        ],
        out_specs=[_rowblk(RN, 1)],
        out_shape=[jax.ShapeDtypeStruct((B * K_NEG, 1), f32)],
    )(gn_mem, gn_emb, tenc, u, bbilr, *wargs)[0]

    return (pos.reshape(B), negs.reshape(B, K_NEG),
            heads[:, 0:3], heads[:, 3:6], heads[:, 6:9])


# speedup vs baseline: 1.4633x; 1.0642x over previous
import functools
import math

import jax
import jax.numpy as jnp
from jax import lax
from jax.experimental import pallas as pl
from jax.experimental.pallas import tpu as pltpu
from jax.experimental.pallas import tpu_sc as plsc

NUM_NODES = 100000
D_MODEL = 128
LLM_DIM = 768
B = 4096
K_NEG = 16
TOT = B + B + B * K_NEG


@functools.cache
def _make_sc_gather(tot, chunk):
    nc, ns = 2, 16
    nw = nc * ns
    rows_w = tot // nw
    n_chunks = rows_w // chunk

    mesh = plsc.VectorSubcoreMesh(core_axis_name="c", subcore_axis_name="s")

    @functools.partial(
        pl.kernel,
        mesh=mesh,
        out_type=jax.ShapeDtypeStruct((tot, D_MODEL), jnp.float32),
        scratch_types=[
            pltpu.VMEM((rows_w,), jnp.int32),
            pltpu.VMEM((chunk, D_MODEL), jnp.float32),
            pltpu.VMEM((chunk, D_MODEL), jnp.float32),
            pltpu.SemaphoreType.DMA,
            pltpu.SemaphoreType.DMA,
            pltpu.SemaphoreType.DMA,
            pltpu.SemaphoreType.DMA,
            pltpu.SemaphoreType.DMA,
            pltpu.SemaphoreType.DMA,
        ],
    )
    def gather_k(mem_hbm, emb_hbm, idx_hbm, out, idx_v, buf0, buf1,
                 g1s0, g1s1, g2s0, g2s1, ws0, ws1):
        wid = lax.axis_index("s") * nc + lax.axis_index("c")
        base = wid * rows_w
        pltpu.sync_copy(idx_hbm.at[pl.ds(base, rows_w)], idx_v)
        bufs = (buf0, buf1)
        g1s, g2s, ws = (g1s0, g1s1), (g2s0, g2s1), (ws0, ws1)

        def g1(c):
            p = c % 2
            iv = idx_v.at[pl.ds(c * chunk, chunk)]
            return pltpu.make_async_copy(mem_hbm.at[iv], bufs[p], g1s[p])

        def g2(c):
            p = c % 2
            iv = idx_v.at[pl.ds(c * chunk, chunk)]
            return pltpu.make_async_copy(emb_hbm.at[iv], bufs[p], g2s[p])

        def w(c):
            p = c % 2
            dst = out.at[pl.ds(base + c * chunk, chunk)]
            return pltpu.make_async_copy(bufs[p], dst, ws[p])

        for t in range(n_chunks + 2):
            c0, c1, c2 = t, t - 1, t - 2
            if 0 <= c2 < n_chunks:
                g2(c2).wait()
                w(c2).start()
            if 0 <= c1 < n_chunks:
                g1(c1).wait()
                g2(c1).start(add=True)
            if c0 < n_chunks:
                if c0 >= 2:
                    w(c0 - 2).wait()
                g1(c0).start()
        for c in range(max(n_chunks - 2, 0), n_chunks):
            w(c).wait()

    return gather_k


def _sc_gather(memory, node_emb, idx, chunk):
    return _make_sc_gather(idx.shape[0], chunk)(memory, node_emb, idx)


_SQRT_2 = math.sqrt(2.0)


def _time_enc_block(ts_col, rows):
    j = lax.broadcasted_iota(jnp.int32, (rows, D_MODEL), 1)
    half = (j // 2).astype(jnp.float32)
    div = jnp.exp(half * (-math.log(10000.0) / 64.0))
    phase = ts_col * div
    return jnp.where(j % 2 == 0, jnp.sin(phase), jnp.cos(phase))


def _ln_rows(x, g, b, eps=1e-5):
    m = jnp.mean(x, axis=-1, keepdims=True)
    v = jnp.mean((x - m) ** 2, axis=-1, keepdims=True)
    return (x - m) * lax.rsqrt(v + eps) * g + b


def _fdot(x, w):
    return jnp.dot(x, w, preferred_element_type=jnp.float32)


def _mlp_rows(x, w1, b1, w2, b2, g1, bb1, w3, b3):
    h = _fdot(x, w1) + b1
    h = 0.5 * h * (1.0 + lax.erf(h / _SQRT_2))
    h = _fdot(h, w2) + b2
    h = _ln_rows(h, g1, bb1)
    h = _fdot(h, w3) + b3
    return x + h


def _hidden_block(g_ref, tenc, w1, b1, w2, b2, g1, bb1, w3, b3,
                  g2, bb2):
    x = g_ref[...] + tenc
    x = _ln_rows(x, g2[...], bb2[...])
    return _mlp_rows(x, w1[...], b1[...], w2[...], b2[...], g1[...], bb1[...],
                     w3[...], b3[...])


def _src_body(g, ts, w1, b1, w2, b2, g1, bb1, w3, b3, g2, bb2,
              wbil, wt, bt, u_out, heads_out, tenc_out, *, rows):
    tenc = _time_enc_block(ts[...], rows)
    tenc_out[...] = tenc
    h = _hidden_block(g, tenc, w1, b1, w2, b2, g1, bb1, w3, b3, g2, bb2)
    u_out[...] = _fdot(h, wbil[...])
    raw = _fdot(h, wt[...]) + bt[...]
    lane = lax.broadcasted_iota(jnp.int32, raw.shape, 1)
    neg_inf = jnp.float32(-jnp.inf)
    wl = jnp.where(lane < 3, raw, neg_inf)
    m = jnp.max(wl, axis=1, keepdims=True)
    e = jnp.where(lane < 3, jnp.exp(raw - m), 0.0)
    w = e / jnp.sum(e, axis=1, keepdims=True)
    sp = jnp.maximum(raw, 0.0) + jnp.log1p(jnp.exp(-jnp.abs(raw))) + 1e-6
    heads_out[...] = jnp.where(lane < 3, w, jnp.where(lane < 6, raw, sp))


def _dst_body(g, tenc_ref, u_ref, bbil, w1, b1, w2, b2, g1, bb1, w3, b3,
              g2, bb2, pos_out, *, rows):
    h = _hidden_block(g, tenc_ref[...], w1, b1, w2, b2, g1, bb1, w3, b3,
                      g2, bb2)
    pos_out[...] = jnp.sum(u_ref[...] * h, axis=1, keepdims=True) + bbil[...]


def _rep16(block, nb, rows):
    return jnp.reshape(
        jnp.broadcast_to(block[:, None, :], (nb, K_NEG, D_MODEL)),
        (rows, D_MODEL))


def _neg_body(g, tenc_ref, u_ref, bbil, w1, b1, w2, b2, g1, bb1, w3, b3,
              g2, bb2, neg_out, *, rows):
    nb = rows // K_NEG
    tenc = _rep16(tenc_ref[...], nb, rows)
    h = _hidden_block(g, tenc, w1, b1, w2, b2, g1, bb1, w3, b3, g2, bb2)
    h3 = jnp.reshape(h, (nb, K_NEG, D_MODEL))
    u3 = u_ref[...][:, None, :]
    neg_out[...] = jnp.sum(h3 * u3, axis=2) + bbil[...]


def _full(shape):
    return pl.BlockSpec(shape, lambda i: (0, 0))


def _rowblk(rows, width, off=0):
    return pl.BlockSpec((rows, width), lambda i, off=off: (i + off, 0))


def kernel(node_emb, memory, W1, b1, W2, b2, ln1_g, ln1_b, W3, b3, ln2_g,
           ln2_b, Wbil, bbil, Wt_w, bt_w, Wt_m, bt_m, Wt_s, bt_s, timestamp,
           src, dst, neg_dst):
    f32 = jnp.float32
    src = src.astype(jnp.int32)
    dst = dst.astype(jnp.int32)
    negf = neg_dst.astype(jnp.int32).reshape(-1)

    gn = _sc_gather(memory, node_emb, negf, 256)
    idx_sd = jnp.concatenate([src, dst])
    g_sd = _sc_gather(memory, node_emb, idx_sd, 256)

    ts = timestamp.astype(f32)[:, None]

    wt = jnp.pad(jnp.concatenate([Wt_w, Wt_m, Wt_s], axis=1),
                 ((0, 0), (0, 7)))
    bt = jnp.pad(jnp.concatenate([bt_w, bt_m, bt_s]), (0, 7))[None, :]
    b1r, b2r, b3r = b1[None, :], b2[None, :], b3[None, :]
    g1r, bb1r = ln1_g[None, :], ln1_b[None, :]
    g2r, bb2r = ln2_g[None, :], ln2_b[None, :]
    bbilr = bbil[:, None]

    R = 1024
    wspecs = [
        _full((D_MODEL, LLM_DIM)),
        _full((1, LLM_DIM)),
        _full((LLM_DIM, LLM_DIM)),
        _full((1, LLM_DIM)),
        _full((1, LLM_DIM)),
        _full((1, LLM_DIM)),
        _full((LLM_DIM, D_MODEL)),
        _full((1, D_MODEL)),
        _full((1, D_MODEL)),
        _full((1, D_MODEL)),
    ]
    wargs = (W1, b1r, W2, b2r, g1r, bb1r, W3, b3r, g2r, bb2r)

    u, heads, tenc = pl.pallas_call(
        functools.partial(_src_body, rows=R),
        grid=(B // R,),
        in_specs=[
            _rowblk(R, D_MODEL),
            _rowblk(R, 1),
            *wspecs,
            _full((D_MODEL, D_MODEL)),
            _full((D_MODEL, 16)),
            _full((1, 16)),
        ],
        out_specs=[_rowblk(R, D_MODEL), _rowblk(R, 16), _rowblk(R, D_MODEL)],
        out_shape=[
            jax.ShapeDtypeStruct((B, D_MODEL), f32),
            jax.ShapeDtypeStruct((B, 16), f32),
            jax.ShapeDtypeStruct((B, D_MODEL), f32),
        ],
    )(g_sd, ts, *wargs, Wbil, wt, bt)

    pos = pl.pallas_call(
        functools.partial(_dst_body, rows=R),
        grid=(B // R,),
        in_specs=[
            _rowblk(R, D_MODEL, off=B // R),
            _rowblk(R, D_MODEL),
            _rowblk(R, D_MODEL),
            _full((1, 1)),
            *wspecs,
        ],
        out_specs=[_rowblk(R, 1)],
        out_shape=[jax.ShapeDtypeStruct((B, 1), f32)],
    )(g_sd, tenc, u, bbilr, *wargs)[0]

    RN = 4096
    nb = RN // K_NEG
    negs = pl.pallas_call(
        functools.partial(_neg_body, rows=RN),
        grid=(B * K_NEG // RN,),
        in_specs=[
            _rowblk(RN, D_MODEL),
            pl.BlockSpec((nb, D_MODEL), lambda i: (i, 0)),
            pl.BlockSpec((nb, D_MODEL), lambda i: (i, 0)),
            _full((1, 1)),
            *wspecs,
        ],
        out_specs=[pl.BlockSpec((nb, K_NEG), lambda i: (i, 0))],
        out_shape=[jax.ShapeDtypeStruct((B, K_NEG), f32)],
    )(gn, tenc, u, bbilr, *wargs)[0]

    return (pos.reshape(B), negs,
            heads[:, 0:3], heads[:, 3:6], heads[:, 6:9])


# speedup vs baseline: 1.5304x; 1.0458x over previous
import functools
import math

import jax
import jax.numpy as jnp
from jax import lax
from jax.experimental import pallas as pl
from jax.experimental.pallas import tpu as pltpu
from jax.experimental.pallas import tpu_sc as plsc

NUM_NODES = 100000
D_MODEL = 128
LLM_DIM = 768
B = 4096
K_NEG = 16
TOT = B + B + B * K_NEG


@functools.cache
def _make_sc_gather(tot, chunk):
    nc, ns = 2, 16
    nw = nc * ns
    rows_w = tot // nw
    n_chunks = rows_w // chunk

    mesh = plsc.VectorSubcoreMesh(core_axis_name="c", subcore_axis_name="s")

    @functools.partial(
        pl.kernel,
        mesh=mesh,
        out_type=jax.ShapeDtypeStruct((tot, D_MODEL), jnp.float32),
        scratch_types=[
            pltpu.VMEM((rows_w,), jnp.int32),
            pltpu.VMEM((chunk, D_MODEL), jnp.float32),
            pltpu.VMEM((chunk, D_MODEL), jnp.float32),
            pltpu.SemaphoreType.DMA,
            pltpu.SemaphoreType.DMA,
            pltpu.SemaphoreType.DMA,
            pltpu.SemaphoreType.DMA,
            pltpu.SemaphoreType.DMA,
            pltpu.SemaphoreType.DMA,
        ],
    )
    def gather_k(mem_hbm, emb_hbm, idx_hbm, out, idx_v, buf0, buf1,
                 g1s0, g1s1, g2s0, g2s1, ws0, ws1):
        wid = lax.axis_index("s") * nc + lax.axis_index("c")
        base = wid * rows_w
        pltpu.sync_copy(idx_hbm.at[pl.ds(base, rows_w)], idx_v)
        bufs = (buf0, buf1)
        g1s, g2s, ws = (g1s0, g1s1), (g2s0, g2s1), (ws0, ws1)

        def g1(c):
            p = c % 2
            iv = idx_v.at[pl.ds(c * chunk, chunk)]
            return pltpu.make_async_copy(mem_hbm.at[iv], bufs[p], g1s[p])

        def g2(c):
            p = c % 2
            iv = idx_v.at[pl.ds(c * chunk, chunk)]
            return pltpu.make_async_copy(emb_hbm.at[iv], bufs[p], g2s[p])

        def w(c):
            p = c % 2
            dst = out.at[pl.ds(base + c * chunk, chunk)]
            return pltpu.make_async_copy(bufs[p], dst, ws[p])

        for t in range(n_chunks + 2):
            c0, c1, c2 = t, t - 1, t - 2
            if 0 <= c2 < n_chunks:
                g2(c2).wait()
                w(c2).start()
            if 0 <= c1 < n_chunks:
                g1(c1).wait()
                g2(c1).start(add=True)
            if c0 < n_chunks:
                if c0 >= 2:
                    w(c0 - 2).wait()
                g1(c0).start()
        for c in range(max(n_chunks - 2, 0), n_chunks):
            w(c).wait()

    return gather_k


def _sc_gather(memory, node_emb, idx, chunk):
    return _make_sc_gather(idx.shape[0], chunk)(memory, node_emb, idx)


_SQRT_2 = math.sqrt(2.0)


def _time_enc_block(ts_col, rows):
    j = lax.broadcasted_iota(jnp.int32, (rows, D_MODEL), 1)
    half = (j // 2).astype(jnp.float32)
    div = jnp.exp(half * (-math.log(10000.0) / 64.0))
    phase = ts_col * div
    return jnp.where(j % 2 == 0, jnp.sin(phase), jnp.cos(phase))


def _ln_rows(x, g, b, eps=1e-5):
    m = jnp.mean(x, axis=-1, keepdims=True)
    v = jnp.mean((x - m) ** 2, axis=-1, keepdims=True)
    return (x - m) * lax.rsqrt(v + eps) * g + b


def _fdot(x, w):
    return jnp.dot(x, w, preferred_element_type=jnp.float32)


def _mlp_rows(x, w1, b1, w2, b2, g1, bb1, w3, b3):
    h = _fdot(x, w1) + b1
    h = 0.5 * h * (1.0 + lax.erf(h / _SQRT_2))
    h = _fdot(h, w2) + b2
    h = _ln_rows(h, g1, bb1)
    h = _fdot(h, w3) + b3
    return x + h


def _hidden_block(g_ref, tenc, w1, b1, w2, b2, g1, bb1, w3, b3,
                  g2, bb2):
    x = g_ref[...] + tenc
    x = _ln_rows(x, g2[...], bb2[...])
    return _mlp_rows(x, w1[...], b1[...], w2[...], b2[...], g1[...], bb1[...],
                     w3[...], b3[...])


def _srcdst_body(gs, gd, ts, bbil, w1, b1, w2, b2, g1, bb1, w3, b3, g2, bb2,
                 wbil, wt, bt, u_out, heads_out, tenc_out, pos_out, *, rows):
    tenc = _time_enc_block(ts[...], rows)
    tenc_out[...] = tenc
    h = _hidden_block(gs, tenc, w1, b1, w2, b2, g1, bb1, w3, b3, g2, bb2)
    u = _fdot(h, wbil[...])
    u_out[...] = u
    raw = _fdot(h, wt[...]) + bt[...]
    lane = lax.broadcasted_iota(jnp.int32, raw.shape, 1)
    neg_inf = jnp.float32(-jnp.inf)
    wl = jnp.where(lane < 3, raw, neg_inf)
    m = jnp.max(wl, axis=1, keepdims=True)
    e = jnp.where(lane < 3, jnp.exp(raw - m), 0.0)
    w = e / jnp.sum(e, axis=1, keepdims=True)
    sp = jnp.maximum(raw, 0.0) + jnp.log1p(jnp.exp(-jnp.abs(raw))) + 1e-6
    heads_out[...] = jnp.where(lane < 3, w, jnp.where(lane < 6, raw, sp))
    hd = _hidden_block(gd, tenc, w1, b1, w2, b2, g1, bb1, w3, b3, g2, bb2)
    pos_out[...] = jnp.sum(u * hd, axis=1, keepdims=True) + bbil[...]


def _rep16(block, nb, rows):
    return jnp.reshape(
        jnp.broadcast_to(block[:, None, :], (nb, K_NEG, D_MODEL)),
        (rows, D_MODEL))


def _neg_body(g, tenc_ref, u_ref, bbil, w1, b1, w2, b2, g1, bb1, w3, b3,
              g2, bb2, neg_out, *, rows):
    nb = rows // K_NEG
    tenc = _rep16(tenc_ref[...], nb, rows)
    h = _hidden_block(g, tenc, w1, b1, w2, b2, g1, bb1, w3, b3, g2, bb2)
    h3 = jnp.reshape(h, (nb, K_NEG, D_MODEL))
    u3 = u_ref[...][:, None, :]
    neg_out[...] = jnp.sum(h3 * u3, axis=2) + bbil[...]


def _full(shape):
    return pl.BlockSpec(shape, lambda i: (0, 0))


def _rowblk(rows, width, off=0):
    return pl.BlockSpec((rows, width), lambda i, off=off: (i + off, 0))


def kernel(node_emb, memory, W1, b1, W2, b2, ln1_g, ln1_b, W3, b3, ln2_g,
           ln2_b, Wbil, bbil, Wt_w, bt_w, Wt_m, bt_m, Wt_s, bt_s, timestamp,
           src, dst, neg_dst):
    f32 = jnp.float32
    src = src.astype(jnp.int32)
    dst = dst.astype(jnp.int32)
    negf = neg_dst.astype(jnp.int32).reshape(-1)

    gn = _sc_gather(memory, node_emb, negf, 256)
    idx_sd = jnp.concatenate([src, dst])
    g_sd = _sc_gather(memory, node_emb, idx_sd, 256)

    ts = timestamp.astype(f32)[:, None]

    wt = jnp.pad(jnp.concatenate([Wt_w, Wt_m, Wt_s], axis=1),
                 ((0, 0), (0, 7)))
    bt = jnp.pad(jnp.concatenate([bt_w, bt_m, bt_s]), (0, 7))[None, :]
    b1r, b2r, b3r = b1[None, :], b2[None, :], b3[None, :]
    g1r, bb1r = ln1_g[None, :], ln1_b[None, :]
    g2r, bb2r = ln2_g[None, :], ln2_b[None, :]
    bbilr = bbil[:, None]

    R = 1024
    wspecs = [
        _full((D_MODEL, LLM_DIM)),
        _full((1, LLM_DIM)),
        _full((LLM_DIM, LLM_DIM)),
        _full((1, LLM_DIM)),
        _full((1, LLM_DIM)),
        _full((1, LLM_DIM)),
        _full((LLM_DIM, D_MODEL)),
        _full((1, D_MODEL)),
        _full((1, D_MODEL)),
        _full((1, D_MODEL)),
    ]
    wargs = (W1, b1r, W2, b2r, g1r, bb1r, W3, b3r, g2r, bb2r)

    u, heads, tenc, pos = pl.pallas_call(
        functools.partial(_srcdst_body, rows=R),
        grid=(B // R,),
        in_specs=[
            _rowblk(R, D_MODEL),
            _rowblk(R, D_MODEL, off=B // R),
            _rowblk(R, 1),
            _full((1, 1)),
            *wspecs,
            _full((D_MODEL, D_MODEL)),
            _full((D_MODEL, 16)),
            _full((1, 16)),
        ],
        out_specs=[_rowblk(R, D_MODEL), _rowblk(R, 16), _rowblk(R, D_MODEL),
                   _rowblk(R, 1)],
        out_shape=[
            jax.ShapeDtypeStruct((B, D_MODEL), f32),
            jax.ShapeDtypeStruct((B, 16), f32),
            jax.ShapeDtypeStruct((B, D_MODEL), f32),
            jax.ShapeDtypeStruct((B, 1), f32),
        ],
    )(g_sd, g_sd, ts, bbilr, *wargs, Wbil, wt, bt)

    RN = 4096
    nb = RN // K_NEG
    negs = pl.pallas_call(
        functools.partial(_neg_body, rows=RN),
        grid=(B * K_NEG // RN,),
        in_specs=[
            _rowblk(RN, D_MODEL),
            pl.BlockSpec((nb, D_MODEL), lambda i: (i, 0)),
            pl.BlockSpec((nb, D_MODEL), lambda i: (i, 0)),
            _full((1, 1)),
            *wspecs,
        ],
        out_specs=[pl.BlockSpec((nb, K_NEG), lambda i: (i, 0))],
        out_shape=[jax.ShapeDtypeStruct((B, K_NEG), f32)],
    )(gn, tenc, u, bbilr, *wargs)[0]

    return (pos.reshape(B), negs,
            heads[:, 0:3], heads[:, 3:6], heads[:, 6:9])


# speedup vs baseline: 1.5540x; 1.0154x over previous
import functools
import math

import jax
import jax.numpy as jnp
from jax import lax
from jax.experimental import pallas as pl
from jax.experimental.pallas import tpu as pltpu
from jax.experimental.pallas import tpu_sc as plsc

NUM_NODES = 100000
D_MODEL = 128
LLM_DIM = 768
B = 4096
K_NEG = 16
TOT = B + B + B * K_NEG


@functools.cache
def _make_sc_gather(tot, chunk):
    nc, ns = 2, 16
    nw = nc * ns
    rows_w = tot // nw
    n_chunks = rows_w // chunk

    mesh = plsc.VectorSubcoreMesh(core_axis_name="c", subcore_axis_name="s")

    @functools.partial(
        pl.kernel,
        mesh=mesh,
        out_type=jax.ShapeDtypeStruct((tot, D_MODEL), jnp.float32),
        scratch_types=[
            pltpu.VMEM((rows_w,), jnp.int32),
            pltpu.VMEM((chunk, D_MODEL), jnp.float32),
            pltpu.VMEM((chunk, D_MODEL), jnp.float32),
            pltpu.SemaphoreType.DMA,
            pltpu.SemaphoreType.DMA,
            pltpu.SemaphoreType.DMA,
            pltpu.SemaphoreType.DMA,
            pltpu.SemaphoreType.DMA,
            pltpu.SemaphoreType.DMA,
        ],
    )
    def gather_k(mem_hbm, emb_hbm, idx_hbm, out, idx_v, buf0, buf1,
                 g1s0, g1s1, g2s0, g2s1, ws0, ws1):
        wid = lax.axis_index("s") * nc + lax.axis_index("c")
        base = wid * rows_w
        pltpu.sync_copy(idx_hbm.at[pl.ds(base, rows_w)], idx_v)
        bufs = (buf0, buf1)
        g1s, g2s, ws = (g1s0, g1s1), (g2s0, g2s1), (ws0, ws1)

        def g1(c):
            p = c % 2
            iv = idx_v.at[pl.ds(c * chunk, chunk)]
            return pltpu.make_async_copy(mem_hbm.at[iv], bufs[p], g1s[p])

        def g2(c):
            p = c % 2
            iv = idx_v.at[pl.ds(c * chunk, chunk)]
            return pltpu.make_async_copy(emb_hbm.at[iv], bufs[p], g2s[p])

        def w(c):
            p = c % 2
            dst = out.at[pl.ds(base + c * chunk, chunk)]
            return pltpu.make_async_copy(bufs[p], dst, ws[p])

        for t in range(n_chunks + 2):
            c0, c1, c2 = t, t - 1, t - 2
            if 0 <= c2 < n_chunks:
                g2(c2).wait()
                w(c2).start()
            if 0 <= c1 < n_chunks:
                g1(c1).wait()
                g2(c1).start(add=True)
            if c0 < n_chunks:
                if c0 >= 2:
                    w(c0 - 2).wait()
                g1(c0).start()
        for c in range(max(n_chunks - 2, 0), n_chunks):
            w(c).wait()

    return gather_k


def _sc_gather(memory, node_emb, idx, chunk):
    return _make_sc_gather(idx.shape[0], chunk)(memory, node_emb, idx)


_SQRT_2 = math.sqrt(2.0)


def _time_enc_block(ts_col, rows):
    j = lax.broadcasted_iota(jnp.int32, (rows, D_MODEL), 1)
    half = (j // 2).astype(jnp.float32)
    div = jnp.exp(half * (-math.log(10000.0) / 64.0))
    phase = ts_col * div
    return jnp.where(j % 2 == 0, jnp.sin(phase), jnp.cos(phase))


def _ln_rows(x, g, b, eps=1e-5):
    m = jnp.mean(x, axis=-1, keepdims=True)
    v = jnp.mean((x - m) ** 2, axis=-1, keepdims=True)
    return (x - m) * lax.rsqrt(v + eps) * g + b


def _fdot(x, w):
    return jnp.dot(x, w, preferred_element_type=jnp.float32)


def _mlp_rows(x, w1, b1, w2, b2, g1, bb1, w3, b3):
    h = _fdot(x, w1) + b1
    h = 0.5 * h * (1.0 + lax.erf(h / _SQRT_2))
    h = _fdot(h, w2) + b2
    h = _ln_rows(h, g1, bb1)
    h = _fdot(h, w3) + b3
    return x + h


def _hidden_block(g_ref, tenc, w1, b1, w2, b2, g1, bb1, w3, b3,
                  g2, bb2):
    x = g_ref[...] + tenc
    x = _ln_rows(x, g2[...], bb2[...])
    return _mlp_rows(x, w1[...], b1[...], w2[...], b2[...], g1[...], bb1[...],
                     w3[...], b3[...])


def _srcdst_body(gs, gd, ts, bbil, w1, b1, w2, b2, g1, bb1, w3, b3, g2, bb2,
                 wbil, wt, bt, u_out, weights_out, means_out, stds_out,
                 tenc_out, pos_out, *, rows):
    tenc = _time_enc_block(ts[...], rows)
    tenc_out[...] = tenc
    h = _hidden_block(gs, tenc, w1, b1, w2, b2, g1, bb1, w3, b3, g2, bb2)
    u = _fdot(h, wbil[...])
    u_out[...] = u
    raw = _fdot(h, wt[...]) + bt[...]
    lane = lax.broadcasted_iota(jnp.int32, raw.shape, 1)
    neg_inf = jnp.float32(-jnp.inf)
    wl = jnp.where(lane < 3, raw, neg_inf)
    m = jnp.max(wl, axis=1, keepdims=True)
    e = jnp.where(lane < 3, jnp.exp(raw - m), 0.0)
    w = e / jnp.sum(e, axis=1, keepdims=True)
    sp = jnp.maximum(raw, 0.0) + jnp.log1p(jnp.exp(-jnp.abs(raw))) + 1e-6
    weights_out[...] = w[:, 0:3]
    means_out[...] = raw[:, 3:6]
    stds_out[...] = sp[:, 6:9]
    hd = _hidden_block(gd, tenc, w1, b1, w2, b2, g1, bb1, w3, b3, g2, bb2)
    pos_out[...] = jnp.sum(u * hd, axis=1, keepdims=True) + bbil[...]


def _rep16(block, nb, rows):
    return jnp.reshape(
        jnp.broadcast_to(block[:, None, :], (nb, K_NEG, D_MODEL)),
        (rows, D_MODEL))


def _neg_body(g, tenc_ref, u_ref, bbil, w1, b1, w2, b2, g1, bb1, w3, b3,
              g2, bb2, neg_out, *, rows):
    nb = rows // K_NEG
    tenc = _rep16(tenc_ref[...], nb, rows)
    h = _hidden_block(g, tenc, w1, b1, w2, b2, g1, bb1, w3, b3, g2, bb2)
    h3 = jnp.reshape(h, (nb, K_NEG, D_MODEL))
    u3 = u_ref[...][:, None, :]
    neg_out[...] = jnp.sum(h3 * u3, axis=2) + bbil[...]


def _full(shape):
    return pl.BlockSpec(shape, lambda i: (0, 0))


def _rowblk(rows, width, off=0):
    return pl.BlockSpec((rows, width), lambda i, off=off: (i + off, 0))


def kernel(node_emb, memory, W1, b1, W2, b2, ln1_g, ln1_b, W3, b3, ln2_g,
           ln2_b, Wbil, bbil, Wt_w, bt_w, Wt_m, bt_m, Wt_s, bt_s, timestamp,
           src, dst, neg_dst):
    f32 = jnp.float32
    src = src.astype(jnp.int32)
    dst = dst.astype(jnp.int32)
    negf = neg_dst.astype(jnp.int32).reshape(-1)

    gn = _sc_gather(memory, node_emb, negf, 256)
    idx_sd = jnp.concatenate([src, dst])
    g_sd = _sc_gather(memory, node_emb, idx_sd, 256)

    ts = timestamp.astype(f32)[:, None]

    wt = jnp.pad(jnp.concatenate([Wt_w, Wt_m, Wt_s], axis=1),
                 ((0, 0), (0, 7)))
    bt = jnp.pad(jnp.concatenate([bt_w, bt_m, bt_s]), (0, 7))[None, :]
    b1r, b2r, b3r = b1[None, :], b2[None, :], b3[None, :]
    g1r, bb1r = ln1_g[None, :], ln1_b[None, :]
    g2r, bb2r = ln2_g[None, :], ln2_b[None, :]
    bbilr = bbil[:, None]

    R = 1024
    wspecs = [
        _full((D_MODEL, LLM_DIM)),
        _full((1, LLM_DIM)),
        _full((LLM_DIM, LLM_DIM)),
        _full((1, LLM_DIM)),
        _full((1, LLM_DIM)),
        _full((1, LLM_DIM)),
        _full((LLM_DIM, D_MODEL)),
        _full((1, D_MODEL)),
        _full((1, D_MODEL)),
        _full((1, D_MODEL)),
    ]
    wargs = (W1, b1r, W2, b2r, g1r, bb1r, W3, b3r, g2r, bb2r)

    u, wgt, mns, sds, tenc, pos = pl.pallas_call(
        functools.partial(_srcdst_body, rows=R),
        grid=(B // R,),
        in_specs=[
            _rowblk(R, D_MODEL),
            _rowblk(R, D_MODEL, off=B // R),
            _rowblk(R, 1),
            _full((1, 1)),
            *wspecs,
            _full((D_MODEL, D_MODEL)),
            _full((D_MODEL, 16)),
            _full((1, 16)),
        ],
        out_specs=[_rowblk(R, D_MODEL), _rowblk(R, 3), _rowblk(R, 3),
                   _rowblk(R, 3), _rowblk(R, D_MODEL), _rowblk(R, 1)],
        out_shape=[
            jax.ShapeDtypeStruct((B, D_MODEL), f32),
            jax.ShapeDtypeStruct((B, 3), f32),
            jax.ShapeDtypeStruct((B, 3), f32),
            jax.ShapeDtypeStruct((B, 3), f32),
            jax.ShapeDtypeStruct((B, D_MODEL), f32),
            jax.ShapeDtypeStruct((B, 1), f32),
        ],
    )(g_sd, g_sd, ts, bbilr, *wargs, Wbil, wt, bt)

    RN = 4096
    nb = RN // K_NEG
    negs = pl.pallas_call(
        functools.partial(_neg_body, rows=RN),
        grid=(B * K_NEG // RN,),
        in_specs=[
            _rowblk(RN, D_MODEL),
            pl.BlockSpec((nb, D_MODEL), lambda i: (i, 0)),
            pl.BlockSpec((nb, D_MODEL), lambda i: (i, 0)),
            _full((1, 1)),
            *wspecs,
        ],
        out_specs=[pl.BlockSpec((nb, K_NEG), lambda i: (i, 0))],
        out_shape=[jax.ShapeDtypeStruct((B, K_NEG), f32)],
    )(gn, tenc, u, bbilr, *wargs)[0]

    return (pos.reshape(B), negs, wgt, mns, sds)


# speedup vs baseline: 1.5544x; 1.0003x over previous
import functools
import math

import jax
import jax.numpy as jnp
from jax import lax
from jax.experimental import pallas as pl
from jax.experimental.pallas import tpu as pltpu
from jax.experimental.pallas import tpu_sc as plsc

NUM_NODES = 100000
D_MODEL = 128
LLM_DIM = 768
B = 4096
K_NEG = 16
TOT = B + B + B * K_NEG


@functools.cache
def _make_sc_gather(tot, chunk):
    nc, ns = 2, 16
    nw = nc * ns
    rows_w = tot // nw
    n_chunks = rows_w // chunk

    mesh = plsc.VectorSubcoreMesh(core_axis_name="c", subcore_axis_name="s")

    @functools.partial(
        pl.kernel,
        mesh=mesh,
        out_type=jax.ShapeDtypeStruct((tot, D_MODEL), jnp.float32),
        scratch_types=[
            pltpu.VMEM((rows_w,), jnp.int32),
            pltpu.VMEM((chunk, D_MODEL), jnp.float32),
            pltpu.VMEM((chunk, D_MODEL), jnp.float32),
            pltpu.SemaphoreType.DMA,
            pltpu.SemaphoreType.DMA,
            pltpu.SemaphoreType.DMA,
            pltpu.SemaphoreType.DMA,
            pltpu.SemaphoreType.DMA,
            pltpu.SemaphoreType.DMA,
        ],
    )
    def gather_k(mem_hbm, emb_hbm, idx_hbm, out, idx_v, buf0, buf1,
                 g1s0, g1s1, g2s0, g2s1, ws0, ws1):
        wid = lax.axis_index("s") * nc + lax.axis_index("c")
        base = wid * rows_w
        pltpu.sync_copy(idx_hbm.at[pl.ds(base, rows_w)], idx_v)
        bufs = (buf0, buf1)
        g1s, g2s, ws = (g1s0, g1s1), (g2s0, g2s1), (ws0, ws1)

        def g1(c):
            p = c % 2
            iv = idx_v.at[pl.ds(c * chunk, chunk)]
            return pltpu.make_async_copy(mem_hbm.at[iv], bufs[p], g1s[p])

        def g2(c):
            p = c % 2
            iv = idx_v.at[pl.ds(c * chunk, chunk)]
            return pltpu.make_async_copy(emb_hbm.at[iv], bufs[p], g2s[p])

        def w(c):
            p = c % 2
            dst = out.at[pl.ds(base + c * chunk, chunk)]
            return pltpu.make_async_copy(bufs[p], dst, ws[p])

        for t in range(n_chunks + 2):
            c0, c1, c2 = t, t - 1, t - 2
            if 0 <= c2 < n_chunks:
                g2(c2).wait()
                w(c2).start()
            if 0 <= c1 < n_chunks:
                g1(c1).wait()
                g2(c1).start(add=True)
            if c0 < n_chunks:
                if c0 >= 2:
                    w(c0 - 2).wait()
                g1(c0).start()
        for c in range(max(n_chunks - 2, 0), n_chunks):
            w(c).wait()

    return gather_k


def _sc_gather(memory, node_emb, idx, chunk):
    return _make_sc_gather(idx.shape[0], chunk)(memory, node_emb, idx)


_SQRT_2 = math.sqrt(2.0)


def _time_enc_block(ts_col, rows):
    j = lax.broadcasted_iota(jnp.int32, (rows, D_MODEL), 1)
    half = (j // 2).astype(jnp.float32)
    div = jnp.exp(half * (-math.log(10000.0) / 64.0))
    phase = ts_col * div
    return jnp.where(j % 2 == 0, jnp.sin(phase), jnp.cos(phase))


def _ln_rows(x, g, b, eps=1e-5):
    m = jnp.mean(x, axis=-1, keepdims=True)
    v = jnp.mean((x - m) ** 2, axis=-1, keepdims=True)
    return (x - m) * lax.rsqrt(v + eps) * g + b


def _fdot(x, w):
    return jnp.dot(x, w, preferred_element_type=jnp.float32)


def _mlp_rows(x, w1, b1, w2, b2, g1, bb1, w3, b3):
    h = _fdot(x, w1) + b1
    h = 0.5 * h * (1.0 + lax.erf(h / _SQRT_2))
    h = _fdot(h, w2) + b2
    h = _ln_rows(h, g1, bb1)
    h = _fdot(h, w3) + b3
    return x + h


def _hidden_block(g_ref, tenc, w1, b1, w2, b2, g1, bb1, w3, b3,
                  g2, bb2):
    x = g_ref[...] + tenc
    x = _ln_rows(x, g2[...], bb2[...])
    return _mlp_rows(x, w1[...], b1[...], w2[...], b2[...], g1[...], bb1[...],
                     w3[...], b3[...])


def _srcdst_body(gs, gd, ts, bbil, w1, b1, w2, b2, g1, bb1, w3, b3, g2, bb2,
                 wbil, wt, bt, u_out, weights_out, means_out, stds_out,
                 tenc_out, pos_out, *, rows):
    tenc = _time_enc_block(ts[...], rows)
    tenc_out[...] = tenc
    h = _hidden_block(gs, tenc, w1, b1, w2, b2, g1, bb1, w3, b3, g2, bb2)
    u = _fdot(h, wbil[...])
    u_out[...] = u
    raw = _fdot(h, wt[...]) + bt[...]
    lane = lax.broadcasted_iota(jnp.int32, raw.shape, 1)
    neg_inf = jnp.float32(-jnp.inf)
    wl = jnp.where(lane < 3, raw, neg_inf)
    m = jnp.max(wl, axis=1, keepdims=True)
    e = jnp.where(lane < 3, jnp.exp(raw - m), 0.0)
    w = e / jnp.sum(e, axis=1, keepdims=True)
    sp = jnp.maximum(raw, 0.0) + jnp.log1p(jnp.exp(-jnp.abs(raw))) + 1e-6
    weights_out[...] = w[:, 0:3]
    means_out[...] = raw[:, 3:6]
    stds_out[...] = sp[:, 6:9]
    hd = _hidden_block(gd, tenc, w1, b1, w2, b2, g1, bb1, w3, b3, g2, bb2)
    pos_out[...] = jnp.sum(u * hd, axis=1, keepdims=True) + bbil[...]


def _rep16(block, nb, rows):
    return jnp.reshape(
        jnp.broadcast_to(block[:, None, :], (nb, K_NEG, D_MODEL)),
        (rows, D_MODEL))


def _neg_body(g, tenc_ref, u_ref, bbil, w1, b1, w2, b2, g1, bb1, w3, b3,
              g2, bb2, neg_out, *, rows):
    nb = rows // K_NEG
    tenc = _rep16(tenc_ref[...], nb, rows)
    h = _hidden_block(g, tenc, w1, b1, w2, b2, g1, bb1, w3, b3, g2, bb2)
    h3 = jnp.reshape(h, (nb, K_NEG, D_MODEL))
    u3 = u_ref[...][:, None, :]
    neg_out[...] = jnp.sum(h3 * u3, axis=2) + bbil[...]


def _full(shape):
    return pl.BlockSpec(shape, lambda i: (0, 0))


def _rowblk(rows, width, off=0):
    return pl.BlockSpec((rows, width), lambda i, off=off: (i + off, 0))


def kernel(node_emb, memory, W1, b1, W2, b2, ln1_g, ln1_b, W3, b3, ln2_g,
           ln2_b, Wbil, bbil, Wt_w, bt_w, Wt_m, bt_m, Wt_s, bt_s, timestamp,
           src, dst, neg_dst):
    f32 = jnp.float32
    src = src.astype(jnp.int32)
    dst = dst.astype(jnp.int32)
    negf = neg_dst.astype(jnp.int32).reshape(-1)

    gn = _sc_gather(memory, node_emb, negf, 256)
    idx_sd = jnp.concatenate([src, dst])
    g_sd = _sc_gather(memory, node_emb, idx_sd, 256)

    ts = timestamp.astype(f32)[:, None]

    wt = jnp.pad(jnp.concatenate([Wt_w, Wt_m, Wt_s], axis=1),
                 ((0, 0), (0, 7)))
    bt = jnp.pad(jnp.concatenate([bt_w, bt_m, bt_s]), (0, 7))[None, :]
    b1r, b2r, b3r = b1[None, :], b2[None, :], b3[None, :]
    g1r, bb1r = ln1_g[None, :], ln1_b[None, :]
    g2r, bb2r = ln2_g[None, :], ln2_b[None, :]
    bbilr = bbil[:, None]

    R = 1024
    wspecs = [
        _full((D_MODEL, LLM_DIM)),
        _full((1, LLM_DIM)),
        _full((LLM_DIM, LLM_DIM)),
        _full((1, LLM_DIM)),
        _full((1, LLM_DIM)),
        _full((1, LLM_DIM)),
        _full((LLM_DIM, D_MODEL)),
        _full((1, D_MODEL)),
        _full((1, D_MODEL)),
        _full((1, D_MODEL)),
    ]
    wargs = (W1, b1r, W2, b2r, g1r, bb1r, W3, b3r, g2r, bb2r)

    u, wgt, mns, sds, tenc, pos = pl.pallas_call(
        functools.partial(_srcdst_body, rows=R),
        grid=(B // R,),
        compiler_params=pltpu.CompilerParams(
            dimension_semantics=("parallel",)),
        in_specs=[
            _rowblk(R, D_MODEL),
            _rowblk(R, D_MODEL, off=B // R),
            _rowblk(R, 1),
            _full((1, 1)),
            *wspecs,
            _full((D_MODEL, D_MODEL)),
            _full((D_MODEL, 16)),
            _full((1, 16)),
        ],
        out_specs=[_rowblk(R, D_MODEL), _rowblk(R, 3), _rowblk(R, 3),
                   _rowblk(R, 3), _rowblk(R, D_MODEL), _rowblk(R, 1)],
        out_shape=[
            jax.ShapeDtypeStruct((B, D_MODEL), f32),
            jax.ShapeDtypeStruct((B, 3), f32),
            jax.ShapeDtypeStruct((B, 3), f32),
            jax.ShapeDtypeStruct((B, 3), f32),
            jax.ShapeDtypeStruct((B, D_MODEL), f32),
            jax.ShapeDtypeStruct((B, 1), f32),
        ],
    )(g_sd, g_sd, ts, bbilr, *wargs, Wbil, wt, bt)

    RN = 4096
    nb = RN // K_NEG
    negs = pl.pallas_call(
        functools.partial(_neg_body, rows=RN),
        grid=(B * K_NEG // RN,),
        compiler_params=pltpu.CompilerParams(
            dimension_semantics=("parallel",)),
        in_specs=[
            _rowblk(RN, D_MODEL),
            pl.BlockSpec((nb, D_MODEL), lambda i: (i, 0)),
            pl.BlockSpec((nb, D_MODEL), lambda i: (i, 0)),
            _full((1, 1)),
            *wspecs,
        ],
        out_specs=[pl.BlockSpec((nb, K_NEG), lambda i: (i, 0))],
        out_shape=[jax.ShapeDtypeStruct((B, K_NEG), f32)],
    )(gn, tenc, u, bbilr, *wargs)[0]

    return (pos.reshape(B), negs, wgt, mns, sds)


# speedup vs baseline: 1.6581x; 1.0667x over previous
import functools
import math

import jax
import jax.numpy as jnp
from jax import lax
from jax.experimental import pallas as pl
from jax.experimental.pallas import tpu as pltpu
from jax.experimental.pallas import tpu_sc as plsc

NUM_NODES = 100000
D_MODEL = 128
LLM_DIM = 768
B = 4096
K_NEG = 16
TOT = B + B + B * K_NEG


@functools.cache
def _make_sc_gather(tot, chunk):
    nc, ns = 2, 16
    nw = nc * ns
    rows_w = tot // nw
    n_chunks = rows_w // chunk

    mesh = plsc.VectorSubcoreMesh(core_axis_name="c", subcore_axis_name="s")

    @functools.partial(
        pl.kernel,
        mesh=mesh,
        out_type=jax.ShapeDtypeStruct((tot, D_MODEL), jnp.float32),
        scratch_types=[
            pltpu.VMEM((rows_w,), jnp.int32),
            pltpu.VMEM((chunk, D_MODEL), jnp.float32),
            pltpu.VMEM((chunk, D_MODEL), jnp.float32),
            pltpu.SemaphoreType.DMA,
            pltpu.SemaphoreType.DMA,
            pltpu.SemaphoreType.DMA,
            pltpu.SemaphoreType.DMA,
            pltpu.SemaphoreType.DMA,
            pltpu.SemaphoreType.DMA,
        ],
    )
    def gather_k(mem_hbm, emb_hbm, idx_hbm, out, idx_v, buf0, buf1,
                 g1s0, g1s1, g2s0, g2s1, ws0, ws1):
        wid = lax.axis_index("s") * nc + lax.axis_index("c")
        base = wid * rows_w
        pltpu.sync_copy(idx_hbm.at[pl.ds(base, rows_w)], idx_v)
        bufs = (buf0, buf1)
        g1s, g2s, ws = (g1s0, g1s1), (g2s0, g2s1), (ws0, ws1)

        def g1(c):
            p = c % 2
            iv = idx_v.at[pl.ds(c * chunk, chunk)]
            return pltpu.make_async_copy(mem_hbm.at[iv], bufs[p], g1s[p])

        def g2(c):
            p = c % 2
            iv = idx_v.at[pl.ds(c * chunk, chunk)]
            return pltpu.make_async_copy(emb_hbm.at[iv], bufs[p], g2s[p])

        def w(c):
            p = c % 2
            dst = out.at[pl.ds(base + c * chunk, chunk)]
            return pltpu.make_async_copy(bufs[p], dst, ws[p])

        for t in range(n_chunks + 2):
            c0, c1, c2 = t, t - 1, t - 2
            if 0 <= c2 < n_chunks:
                g2(c2).wait()
                w(c2).start()
            if 0 <= c1 < n_chunks:
                g1(c1).wait()
                g2(c1).start(add=True)
            if c0 < n_chunks:
                if c0 >= 2:
                    w(c0 - 2).wait()
                g1(c0).start()
        for c in range(max(n_chunks - 2, 0), n_chunks):
            w(c).wait()

    return gather_k


def _sc_gather(memory, node_emb, idx, chunk):
    return _make_sc_gather(idx.shape[0], chunk)(memory, node_emb, idx)


_SQRT_2 = math.sqrt(2.0)


def _time_enc_block(ts_col, rows):
    j = lax.broadcasted_iota(jnp.int32, (rows, D_MODEL), 1)
    half = (j // 2).astype(jnp.float32)
    div = jnp.exp(half * (-math.log(10000.0) / 64.0))
    phase = ts_col * div
    return jnp.where(j % 2 == 0, jnp.sin(phase), jnp.cos(phase))


def _ln_rows(x, g, b, eps=1e-5):
    m = jnp.mean(x, axis=-1, keepdims=True)
    v = jnp.mean((x - m) ** 2, axis=-1, keepdims=True)
    return (x - m) * lax.rsqrt(v + eps) * g + b


def _fdot(x, w):
    return jnp.dot(x, w, preferred_element_type=jnp.float32)


def _mlp_rows(x, w1, b1, w2, b2, g1, bb1, w3, b3):
    h = _fdot(x, w1) + b1
    h = 0.5 * h * (1.0 + lax.erf(h / _SQRT_2))
    h = _fdot(h, w2) + b2
    v = jnp.mean(h * h, axis=-1, keepdims=True)
    h = h * lax.rsqrt(v + 1e-5) * g1 + bb1
    h = _fdot(h, w3) + b3
    return x + h


def _hidden_block(g_ref, tenc, w1, b1, w2, b2, g1, bb1, w3, b3,
                  g2, bb2):
    x = g_ref[...] + tenc
    x = _ln_rows(x, g2[...], bb2[...])
    return _mlp_rows(x, w1[...], b1[...], w2[...], b2[...], g1[...], bb1[...],
                     w3[...], b3[...])


def _srcdst_body(gs, gd, ts, bbil, w1, b1, w2, b2, g1, bb1, w3, b3, g2, bb2,
                 wbil, wt, bt, u_out, weights_out, means_out, stds_out,
                 tenc_out, pos_out, *, rows):
    tenc = _time_enc_block(ts[...], rows)
    tenc_out[...] = tenc
    h = _hidden_block(gs, tenc, w1, b1, w2, b2, g1, bb1, w3, b3, g2, bb2)
    u = _fdot(h, wbil[...])
    u_out[...] = u
    raw = _fdot(h, wt[...]) + bt[...]
    lane = lax.broadcasted_iota(jnp.int32, raw.shape, 1)
    neg_inf = jnp.float32(-jnp.inf)
    wl = jnp.where(lane < 3, raw, neg_inf)
    m = jnp.max(wl, axis=1, keepdims=True)
    e = jnp.where(lane < 3, jnp.exp(raw - m), 0.0)
    w = e / jnp.sum(e, axis=1, keepdims=True)
    sp = jnp.maximum(raw, 0.0) + jnp.log1p(jnp.exp(-jnp.abs(raw))) + 1e-6
    weights_out[...] = w[:, 0:3]
    means_out[...] = raw[:, 3:6]
    stds_out[...] = sp[:, 6:9]
    hd = _hidden_block(gd, tenc, w1, b1, w2, b2, g1, bb1, w3, b3, g2, bb2)
    pos_out[...] = jnp.sum(u * hd, axis=1, keepdims=True) + bbil[...]


def _rep16(block, nb, rows):
    return jnp.reshape(
        jnp.broadcast_to(block[:, None, :], (nb, K_NEG, D_MODEL)),
        (rows, D_MODEL))


def _neg_body(g, tenc_ref, u_ref, bbil, w1, b1, w2, b2, g1, bb1, w3, b3,
              g2, bb2, neg_out, *, rows):
    nb = rows // K_NEG
    tenc = _rep16(tenc_ref[...], nb, rows)
    h = _hidden_block(g, tenc, w1, b1, w2, b2, g1, bb1, w3, b3, g2, bb2)
    h3 = jnp.reshape(h, (nb, K_NEG, D_MODEL))
    u3 = u_ref[...][:, None, :]
    neg_out[...] = jnp.sum(h3 * u3, axis=2) + bbil[...]


def _full(shape):
    return pl.BlockSpec(shape, lambda i: (0, 0))


def _rowblk(rows, width, off=0):
    return pl.BlockSpec((rows, width), lambda i, off=off: (i + off, 0))


def kernel(node_emb, memory, W1, b1, W2, b2, ln1_g, ln1_b, W3, b3, ln2_g,
           ln2_b, Wbil, bbil, Wt_w, bt_w, Wt_m, bt_m, Wt_s, bt_s, timestamp,
           src, dst, neg_dst):
    f32 = jnp.float32
    src = src.astype(jnp.int32)
    dst = dst.astype(jnp.int32)
    negf = neg_dst.astype(jnp.int32).reshape(-1)

    gn = _sc_gather(memory, node_emb, negf, 256)
    idx_sd = jnp.concatenate([src, dst])
    g_sd = _sc_gather(memory, node_emb, idx_sd, 256)

    ts = timestamp.astype(f32)[:, None]

    wt = jnp.pad(jnp.concatenate([Wt_w, Wt_m, Wt_s], axis=1),
                 ((0, 0), (0, 7)))
    bt = jnp.pad(jnp.concatenate([bt_w, bt_m, bt_s]), (0, 7))[None, :]
    b1r, b2r, b3r = b1[None, :], b2[None, :], b3[None, :]
    g1r, bb1r = ln1_g[None, :], ln1_b[None, :]
    g2r, bb2r = ln2_g[None, :], ln2_b[None, :]
    bbilr = bbil[:, None]

    R = 1024
    wspecs = [
        _full((D_MODEL, LLM_DIM)),
        _full((1, LLM_DIM)),
        _full((LLM_DIM, LLM_DIM)),
        _full((1, LLM_DIM)),
        _full((1, LLM_DIM)),
        _full((1, LLM_DIM)),
        _full((LLM_DIM, D_MODEL)),
        _full((1, D_MODEL)),
        _full((1, D_MODEL)),
        _full((1, D_MODEL)),
    ]
    W2c = W2 - jnp.mean(W2, axis=1, keepdims=True)
    b2c = (b2 - jnp.mean(b2))[None, :]
    wargs = (W1, b1r, W2c, b2c, g1r, bb1r, W3, b3r, g2r, bb2r)

    u, wgt, mns, sds, tenc, pos = pl.pallas_call(
        functools.partial(_srcdst_body, rows=R),
        grid=(B // R,),
        compiler_params=pltpu.CompilerParams(
            dimension_semantics=("parallel",)),
        in_specs=[
            _rowblk(R, D_MODEL),
            _rowblk(R, D_MODEL, off=B // R),
            _rowblk(R, 1),
            _full((1, 1)),
            *wspecs,
            _full((D_MODEL, D_MODEL)),
            _full((D_MODEL, 16)),
            _full((1, 16)),
        ],
        out_specs=[_rowblk(R, D_MODEL), _rowblk(R, 3), _rowblk(R, 3),
                   _rowblk(R, 3), _rowblk(R, D_MODEL), _rowblk(R, 1)],
        out_shape=[
            jax.ShapeDtypeStruct((B, D_MODEL), f32),
            jax.ShapeDtypeStruct((B, 3), f32),
            jax.ShapeDtypeStruct((B, 3), f32),
            jax.ShapeDtypeStruct((B, 3), f32),
            jax.ShapeDtypeStruct((B, D_MODEL), f32),
            jax.ShapeDtypeStruct((B, 1), f32),
        ],
    )(g_sd, g_sd, ts, bbilr, *wargs, Wbil, wt, bt)

    RN = 4096
    nb = RN // K_NEG
    negs = pl.pallas_call(
        functools.partial(_neg_body, rows=RN),
        grid=(B * K_NEG // RN,),
        compiler_params=pltpu.CompilerParams(
            dimension_semantics=("parallel",)),
        in_specs=[
            _rowblk(RN, D_MODEL),
            pl.BlockSpec((nb, D_MODEL), lambda i: (i, 0)),
            pl.BlockSpec((nb, D_MODEL), lambda i: (i, 0)),
            _full((1, 1)),
            *wspecs,
        ],
        out_specs=[pl.BlockSpec((nb, K_NEG), lambda i: (i, 0))],
        out_shape=[jax.ShapeDtypeStruct((B, K_NEG), f32)],
    )(gn, tenc, u, bbilr, *wargs)[0]

    return (pos.reshape(B), negs, wgt, mns, sds)
